# baseline (speedup 1.0000x reference)
import functools
import math

import jax
import jax.numpy as jnp
from jax.experimental import pallas as pl
from jax.experimental.pallas import tpu as pltpu


def _dec_tail_kernel(a_ref, b_ref, qw_ref, d_ref, o_ref):
    se = a_ref[...] + b_ref[...] + qw_ref[...]
    se = jnp.maximum(se, 0.0)
    o_ref[...] = se @ d_ref[...]


def kernel(x, edge_attr, q_Y_sample, adj, t, num_steps, batch,
           em_W, em_b, emo_W, emo_b,
           lin1_W, lin1_b, nn1_W, nn1_b,
           lin2_W, lin2_b, nn2_W, nn2_b,
           lin3_W, lin3_b, nn3_W, nn3_b,
           dec1_W, dec1_b, dec2_W, dec2_b,
           tm1_W, tm1_b, tm2_W, tm2_b):
    N = x.shape[0]
    E = adj.shape[1]
    src, dst = adj[0], adj[1]
    a = edge_attr[:, 0]
    q = q_Y_sample[:, 0]

    tt = t / num_steps * num_steps * 4
    half = 64
    emb = jnp.exp(jnp.arange(half, dtype=jnp.float32) * -(math.log(10000.0) / (half - 1)))
    emb = tt[:, None] * emb[None, :]
    te0 = jnp.concatenate([jnp.sin(emb), jnp.cos(emb)], axis=-1)
    te = (jax.nn.relu(te0 @ tm1_W + tm1_b) @ tm2_W + tm2_b)[0]

    def uvc(lin_W, lin_b):
        u = (em_W @ lin_W[:64]).ravel()
        v = (emo_W @ lin_W[64:]).ravel()
        c = em_b @ lin_W[:64] + emo_b @ lin_W[64:] + lin_b
        return u, v, c

    u1, v1, c1 = uvc(lin1_W, lin1_b)
    u2, v2, c2 = uvc(lin2_W, lin2_b)
    u3, v3, c3 = uvc(lin3_W, lin3_b)

    msg1 = jax.nn.relu(x[src, 0] + a * u1[0] + q * v1[0] + c1[0])
    s = x[:, 0] + jax.ops.segment_sum(msg1, dst, num_segments=N)
    h1 = s[:, None] * nn1_W[0][None, :] + (nn1_b + te)[None, :]

    msg2 = jax.nn.relu(h1[src] + a[:, None] * u2[None, :] + q[:, None] * v2[None, :] + c2[None, :])
    aggr2 = jax.ops.segment_sum(msg2, dst, num_segments=N)
    h2 = (h1 + aggr2) @ nn2_W + (nn2_b + te)[None, :]

    msg3 = jax.nn.relu(h2[src] + a[:, None] * u3[None, :] + q[:, None] * v3[None, :] + c3[None, :])
    aggr3 = jax.ops.segment_sum(msg3, dst, num_segments=N)
    h3 = (h2 + aggr3) @ nn3_W + (nn3_b + te)[None, :]

    A = h3 @ dec1_W[:128] + dec1_b[None, :]
    B = h3 @ dec1_W[128:256]
    w_q = dec1_W[256]

    As = A[src]
    Bs = B[dst]
    qw = q[:, None] * w_q[None, :]

    BLK = 2000
    out = pl.pallas_call(
        _dec_tail_kernel,
        grid=(E // BLK,),
        in_specs=[
            pl.BlockSpec((BLK, 257), lambda i: (i, 0)),
            pl.BlockSpec((BLK, 257), lambda i: (i, 0)),
            pl.BlockSpec((BLK, 257), lambda i: (i, 0)),
            pl.BlockSpec((257, 1), lambda i: (0, 0)),
        ],
        out_specs=pl.BlockSpec((BLK, 1), lambda i: (i, 0)),
        out_shape=jax.ShapeDtypeStruct((E, 1), jnp.float32),
    )(As, Bs, qw, dec2_W)
    return out + dec2_b


# speedup vs baseline: 6.9180x; 6.9180x over previous
import functools
import math

import jax
import jax.numpy as jnp
from jax import lax
from jax.experimental import pallas as pl
from jax.experimental.pallas import tpu as pltpu
from jax.experimental.pallas import tpu_sc as plsc

N = 10000
NP = 10240
E = 320000
H = 128
DD = 256
NC = 2
NS = 16
NW = NC * NS
L = 16
EW = E // NW
CH = 80
NCH = EW // CH
NG = CH // L
RT = NP // NS
NB = 5
BCH = NCH // NB
BE = BCH * CH
NPQ = NP // 4

_mesh = plsc.VectorSubcoreMesh(core_axis_name="c", subcore_axis_name="s",
                               num_cores=NC, num_subcores=NS)
_scp = pltpu.CompilerParams(needs_layout_passes=False)

f32 = jnp.float32
i32 = jnp.int32


def _prep_body(t_ref, ns_ref, em_W, em_b, emo_W, emo_b,
               lin1_W, lin1_b, lin2_W, lin2_b, lin3_W, lin3_b,
               nn1_W, nn1_b, nn2_b, nn3_b,
               tm1_W, tm1_b, tm2_W, tm2_b, dec2_b,
               pk_ref, bt_ref, s16_ref):
    t = t_ref[...]
    ns = ns_ref[...]
    tt = t / ns * ns * 4.0
    idx = lax.broadcasted_iota(i32, (1, 64), 1).astype(f32)
    emb = jnp.exp(idx * (-(math.log(10000.0) / 63.0)))
    emb = tt * emb
    te0 = jnp.concatenate([jnp.sin(emb), jnp.cos(emb)], axis=-1)
    h = jnp.maximum(te0 @ tm1_W[...] + tm1_b[...], 0.0)
    te = h @ tm2_W[...] + tm2_b[...]

    def uvc(lw, lb):
        u = em_W[...] @ lw[:64]
        v = emo_W[...] @ lw[64:]
        c = em_b[...] @ lw[:64] + emo_b[...] @ lw[64:] + lb[...]
        return u, v, c

    u1, v1, c1 = uvc(lin1_W, lin1_b)
    u2, v2, c2 = uvc(lin2_W, lin2_b)
    u3, v3, c3 = uvc(lin3_W, lin3_b)

    b1r = nn1_b[...] + te
    pk_ref[0:1, :] = nn1_W[...]
    pk_ref[1:2, :] = u2
    pk_ref[2:3, :] = v2
    pk_ref[3:4, :] = b1r + c2
    pk_ref[4:5, :] = u3
    pk_ref[5:6, :] = v3
    pk_ref[6:7, :] = c3
    pk_ref[7:8, :] = jnp.zeros((1, H), f32)

    bt_ref[0:1, :] = b1r
    bt_ref[1:2, :] = nn2_b[...] + te
    bt_ref[2:3, :] = nn3_b[...] + te
    bt_ref[3:4, :] = jnp.zeros((1, H), f32)

    ones = jnp.ones((1, 16), f32)
    s16_ref[0:1, :] = u1 * ones
    s16_ref[1:2, :] = v1 * ones
    s16_ref[2:3, :] = c1 * ones
    s16_ref[3:4, :] = dec2_b[...] * ones
    s16_ref[4:8, :] = jnp.zeros((4, 16), f32)


def _s1_body(x_hbm, src_hbm, dst_hbm, a_hbm, q_hbm, s16_hbm,
             out_hbm,
             xbuf, srcp, dstp, abuf, qbuf, aggr, s16v, sumb, res, stage):
    c = lax.axis_index("c")
    s = lax.axis_index("s")
    wid = s * NC + c
    eb = wid * EW
    pltpu.sync_copy(x_hbm, xbuf)
    pltpu.sync_copy(src_hbm.at[pl.ds(eb, EW)], srcp)
    pltpu.sync_copy(dst_hbm.at[pl.ds(eb, EW)], dstp)
    pltpu.sync_copy(a_hbm.at[pl.ds(eb, EW)], abuf)
    pltpu.sync_copy(q_hbm.at[pl.ds(eb, EW)], qbuf)
    pltpu.sync_copy(s16_hbm, s16v)

    zero = jnp.zeros((L,), f32)

    def zbody(i, carry):
        aggr[pl.ds(i * L, L)] = zero
        return carry
    lax.fori_loop(0, NP // L, zbody, 0)

    u1 = s16v[0, :]
    v1 = s16v[1, :]
    c1 = s16v[2, :]

    def grp(g, carry):
        sl = pl.ds(g * L, L)
        src16 = srcp[sl]
        dst16 = dstp[sl]
        x16 = plsc.load_gather(xbuf, [src16])
        a16 = abuf[sl]
        q16 = qbuf[sl]
        m = jnp.maximum(x16 + a16 * u1 + q16 * v1 + c1, 0.0)
        plsc.addupdate_scatter(aggr, [dst16], m)
        return carry
    lax.fori_loop(0, EW // L, grp, 0)

    pltpu.sync_copy(aggr, stage.at[s])
    plsc.subcore_barrier()
    cs = s * RT
    pltpu.sync_copy(stage.at[:, pl.ds(cs, RT)], sumb)

    def rbody(i, carry):
        acc = sumb[0, pl.ds(i * L, L)]
        for r in range(1, NS):
            acc = acc + sumb[r, pl.ds(i * L, L)]
        res[pl.ds(i * L, L)] = acc
        return carry
    lax.fori_loop(0, RT // L, rbody, 0)
    pltpu.sync_copy(res, out_hbm.at[c, pl.ds(cs, RT)])


def _s2_body(x_hbm, p1_hbm, src_hbm, dst_hbm, a_hbm, q_hbm, pk_hbm,
             out_hbm,
             sbuf, tmp, srcp, dstp, dstc, abuf, qbuf, msg, t16, pkv, aggr):
    c = lax.axis_index("c")
    s = lax.axis_index("s")
    wid = s * NC + c
    eb = wid * EW
    pltpu.sync_copy(x_hbm, sbuf)
    for p in range(2):
        for qtr in range(4):
            pltpu.sync_copy(p1_hbm.at[p, pl.ds(qtr * NPQ, NPQ)], tmp)

            def addb(i, carry, _q=qtr):
                sbuf[pl.ds(_q * NPQ + i * L, L)] = (
                    sbuf[pl.ds(_q * NPQ + i * L, L)] + tmp[pl.ds(i * L, L)])
                return carry
            lax.fori_loop(0, NPQ // L, addb, 0)

    pltpu.sync_copy(pk_hbm, pkv)

    zero = jnp.zeros((L,), f32)

    def zb(i, carry):
        for j in range(H // L):
            msg[i, pl.ds(j * L, L)] = zero
        return carry
    lax.fori_loop(0, CH, zb, 0)
    rstart = s * RT
    for b in range(RT // CH):
        pltpu.sync_copy(msg, aggr.at[pl.ds(rstart + b * CH, CH)])
    plsc.subcore_barrier()

    w = [pkv[0, pl.ds(j * L, L)] for j in range(H // L)]
    u = [pkv[1, pl.ds(j * L, L)] for j in range(H // L)]
    v = [pkv[2, pl.ds(j * L, L)] for j in range(H // L)]
    bb = [pkv[3, pl.ds(j * L, L)] for j in range(H // L)]

    for blk in range(NB):
        pltpu.sync_copy(src_hbm.at[pl.ds(eb + blk * BE, BE)], srcp)
        pltpu.sync_copy(dst_hbm.at[pl.ds(eb + blk * BE, BE)], dstp)
        pltpu.sync_copy(a_hbm.at[pl.ds(eb + blk * BE, BE)], abuf)
        pltpu.sync_copy(q_hbm.at[pl.ds(eb + blk * BE, BE)], qbuf)

        def chunk(k, carry):
            def grp(g, carry2):
                sl = pl.ds(k * CH + g * L, L)
                src16 = srcp[sl]
                dstc[pl.ds(g * L, L)] = dstp[sl]
                s16 = plsc.load_gather(sbuf, [src16])
                t16[0, :] = s16
                t16[1, :] = abuf[sl]
                t16[2, :] = qbuf[sl]
                for j in range(L):
                    jidx = jnp.full((L,), j, i32)
                    sv = plsc.load_gather(t16.at[0], [jidx])
                    av = plsc.load_gather(t16.at[1], [jidx])
                    qv = plsc.load_gather(t16.at[2], [jidx])
                    for cg in range(H // L):
                        tt = sv * w[cg] + av * u[cg] + qv * v[cg] + bb[cg]
                        msg[g * L + j, pl.ds(cg * L, L)] = jnp.maximum(tt, 0.0)
                return carry2
            lax.fori_loop(0, NG, grp, 0)
            pltpu.sync_copy(msg, aggr.at[dstc], add=True)
            return carry
        lax.fori_loop(0, BCH, chunk, 0)

    plsc.subcore_barrier()
    pltpu.sync_copy(aggr.at[pl.ds(rstart, RT)],
                    out_hbm.at[c, pl.ds(rstart, RT)])


def _s3_body(h2_hbm, src_hbm, dst_hbm, a_hbm, q_hbm, pk_hbm,
             out_hbm,
             srcp, dstp, srcc, dstc, abuf, qbuf, rows, t16, pkv, aggr, sem):
    c = lax.axis_index("c")
    s = lax.axis_index("s")
    wid = s * NC + c
    eb = wid * EW
    pltpu.sync_copy(pk_hbm, pkv)

    zero = jnp.zeros((L,), f32)

    def zb(i, carry):
        for j in range(H // L):
            rows[i, pl.ds(j * L, L)] = zero
        return carry
    lax.fori_loop(0, CH, zb, 0)
    rstart = s * RT
    for b in range(RT // CH):
        pltpu.sync_copy(rows, aggr.at[pl.ds(rstart + b * CH, CH)])
    plsc.subcore_barrier()

    u = [pkv[4, pl.ds(j * L, L)] for j in range(H // L)]
    v = [pkv[5, pl.ds(j * L, L)] for j in range(H // L)]
    cc = [pkv[6, pl.ds(j * L, L)] for j in range(H // L)]

    for blk in range(NB):
        pltpu.sync_copy(src_hbm.at[pl.ds(eb + blk * BE, BE)], srcp)
        pltpu.sync_copy(dst_hbm.at[pl.ds(eb + blk * BE, BE)], dstp)
        pltpu.sync_copy(a_hbm.at[pl.ds(eb + blk * BE, BE)], abuf)
        pltpu.sync_copy(q_hbm.at[pl.ds(eb + blk * BE, BE)], qbuf)

        def chunk(k, carry):
            def cpy(g, carry2):
                srcc[pl.ds(g * L, L)] = srcp[pl.ds(k * CH + g * L, L)]
                dstc[pl.ds(g * L, L)] = dstp[pl.ds(k * CH + g * L, L)]
                return carry2
            lax.fori_loop(0, NG, cpy, 0)
            pltpu.async_copy(h2_hbm.at[srcc], rows, sem).wait()

            def grp(g, carry2):
                sl = pl.ds(k * CH + g * L, L)
                t16[1, :] = abuf[sl]
                t16[2, :] = qbuf[sl]
                for j in range(L):
                    jidx = jnp.full((L,), j, i32)
                    av = plsc.load_gather(t16.at[1], [jidx])
                    qv = plsc.load_gather(t16.at[2], [jidx])
                    e = g * L + j
                    for cg in range(H // L):
                        slc = pl.ds(cg * L, L)
                        tt = rows[e, slc] + av * u[cg] + qv * v[cg] + cc[cg]
                        rows[e, slc] = jnp.maximum(tt, 0.0)
                return carry2
            lax.fori_loop(0, NG, grp, 0)
            pltpu.sync_copy(rows, aggr.at[dstc], add=True)
            return carry
        lax.fori_loop(0, BCH, chunk, 0)

    plsc.subcore_barrier()
    pltpu.sync_copy(aggr.at[pl.ds(rstart, RT)],
                    out_hbm.at[c, pl.ds(rstart, RT)])


def _s4_body(A_hbm, B_hbm, ab_hbm, src_hbm, dst_hbm, q_hbm, cd_hbm, sc4_hbm,
             out_hbm,
             srcp, dstp, srcc, dstc, qbuf, a256b, b256b, rowsA, rowsB, outb,
             t16, cdv, sc4v, semA, semB):
    c = lax.axis_index("c")
    s = lax.axis_index("s")
    wid = s * NC + c
    eb = wid * EW
    pltpu.sync_copy(src_hbm.at[pl.ds(eb, EW)], srcp)
    pltpu.sync_copy(dst_hbm.at[pl.ds(eb, EW)], dstp)
    pltpu.sync_copy(q_hbm.at[pl.ds(eb, EW)], qbuf)
    pltpu.sync_copy(ab_hbm.at[0], a256b)
    pltpu.sync_copy(ab_hbm.at[1], b256b)
    pltpu.sync_copy(cd_hbm, cdv)
    pltpu.sync_copy(sc4_hbm, sc4v)

    wq = [cdv[0, pl.ds(j * L, L)] for j in range(DD // L)]
    dv = [cdv[1, pl.ds(j * L, L)] for j in range(DD // L)]
    wq256 = sc4v[0, :]
    d256 = sc4v[1, :]
    db2 = sc4v[2, :]
    lane15 = lax.broadcasted_iota(i32, (L,), 0) == (L - 1)

    def chunk(k, carry):
        def cpy(g, carry2):
            srcc[pl.ds(g * L, L)] = srcp[pl.ds(k * CH + g * L, L)]
            dstc[pl.ds(g * L, L)] = dstp[pl.ds(k * CH + g * L, L)]
            return carry2
        lax.fori_loop(0, NG, cpy, 0)
        cpA = pltpu.async_copy(A_hbm.at[srcc], rowsA, semA)
        cpB = pltpu.async_copy(B_hbm.at[dstc], rowsB, semB)
        cpA.wait()
        cpB.wait()

        def grp(g, carry2):
            sl16 = pl.ds(k * CH + g * L, L)
            q16 = qbuf[sl16]
            src16 = srcp[sl16]
            dst16 = dstp[sl16]
            a256 = plsc.load_gather(a256b, [src16])
            b256 = plsc.load_gather(b256b, [dst16])
            v256 = jnp.maximum(a256 + b256 + q16 * wq256, 0.0) * d256
            t16[0, :] = q16
            t16[3, :] = v256
            for j in range(L):
                jidx = jnp.full((L,), j, i32)
                qv = plsc.load_gather(t16.at[0], [jidx])
                vv = plsc.load_gather(t16.at[3], [jidx])
                e = g * L + j
                acc = jnp.zeros((L,), f32)
                for cg in range(DD // L):
                    sl = pl.ds(cg * L, L)
                    tt = jnp.maximum(rowsA[e, sl] + rowsB[e, sl]
                                     + qv * wq[cg], 0.0)
                    acc = acc + tt * dv[cg]
                tot = plsc.cumsum(acc) + (db2 + vv)
                eidx = jnp.full((L,), k * CH, i32) + (g * L + j)
                plsc.store_scatter(outb, [eidx], tot, mask=lane15)
            return carry2
        return lax.fori_loop(0, NG, grp, carry)
    lax.fori_loop(0, NCH, chunk, 0)
    pltpu.sync_copy(outb, out_hbm.at[pl.ds(eb, EW)])


def _t1_body(xc_ref, p1_ref, p2_ref, pk_ref, bt_ref, nn2_ref, h2_ref):
    s_col = xc_ref[...] + p1_ref[:, 0:1] + p1_ref[:, 1:2]
    h1 = s_col * pk_ref[0:1, :] + bt_ref[0:1, :]
    h2in = h1 + p2_ref[0] + p2_ref[1]
    h2_ref[...] = h2in @ nn2_ref[...] + bt_ref[1:2, :]


def _t2_body(h2_ref, p3_ref, bt_ref, nn3_ref, wa_ref, wb_ref, ba_ref,
             wc_ref, bc_ref, a_ref, b_ref, ab_ref):
    h3 = (h2_ref[...] + p3_ref[0] + p3_ref[1]) @ nn3_ref[...] + bt_ref[2:3, :]
    a_ref[...] = h3 @ wa_ref[...] + ba_ref[...]
    b_ref[...] = h3 @ wb_ref[...]
    ab_ref[...] = h3 @ wc_ref[...] + bc_ref[...]


def kernel(x, edge_attr, q_Y_sample, adj, t, num_steps, batch,
           em_W, em_b, emo_W, emo_b,
           lin1_W, lin1_b, nn1_W, nn1_b,
           lin2_W, lin2_b, nn2_W, nn2_b,
           lin3_W, lin3_b, nn3_W, nn3_b,
           dec1_W, dec1_b, dec2_W, dec2_b,
           tm1_W, tm1_b, tm2_W, tm2_b):
    xp = jnp.pad(x[:, 0], (0, NP - N))
    a_e = edge_attr[:, 0]
    q_e = q_Y_sample[:, 0]
    src_e = adj[0]
    dst_e = adj[1]
    t2 = t.reshape(1, 1)
    ns2 = jnp.asarray(num_steps, f32).reshape(1, 1)
    lin1b2 = lin1_b.reshape(1, 1)
    wa_m = dec1_W[:128, :DD]
    wb_m = dec1_W[128:256, :DD]
    ba_m = dec1_b[:DD].reshape(1, DD)
    wc = jnp.stack([dec1_W[:128, DD], dec1_W[128:256, DD]], axis=1)
    bc = jnp.stack([dec1_b[DD], jnp.zeros((), f32)]).reshape(1, 2)
    cd = jnp.stack([dec1_W[256, :DD], dec2_W[:DD, 0]])
    sc4 = jnp.stack([jnp.full((16,), dec1_W[256, DD], f32),
                     jnp.full((16,), dec2_W[DD, 0], f32),
                     jnp.full((16,), dec2_b[0], f32),
                     jnp.zeros((16,), f32)])

    pk, bt, s16 = pl.pallas_call(
        _prep_body,
        out_shape=(jax.ShapeDtypeStruct((8, H), f32),
                   jax.ShapeDtypeStruct((4, H), f32),
                   jax.ShapeDtypeStruct((8, 16), f32)),
    )(t2, ns2, em_W, em_b.reshape(1, 64), emo_W, emo_b.reshape(1, 64),
      lin1_W, lin1b2, lin2_W, lin2_b.reshape(1, H),
      lin3_W, lin3_b.reshape(1, H),
      nn1_W, nn1_b.reshape(1, H), nn2_b.reshape(1, H), nn3_b.reshape(1, H),
      tm1_W, tm1_b.reshape(1, 256), tm2_W, tm2_b.reshape(1, H),
      dec2_b.reshape(1, 1))

    s1 = pl.kernel(
        _s1_body,
        out_type=jax.ShapeDtypeStruct((NC, NP), f32),
        mesh=_mesh,
        compiler_params=_scp,
        scratch_types=[
            pltpu.VMEM((NP,), f32),
            pltpu.VMEM((EW,), i32),
            pltpu.VMEM((EW,), i32),
            pltpu.VMEM((EW,), f32),
            pltpu.VMEM((EW,), f32),
            pltpu.VMEM((NP,), f32),
            pltpu.VMEM((8, 16), f32),
            pltpu.VMEM((NS, RT), f32),
            pltpu.VMEM((RT,), f32),
            pltpu.VMEM_SHARED((NS, NP), f32),
        ],
    )
    part1 = s1(xp, src_e, dst_e, a_e, q_e, s16)

    s2 = pl.kernel(
        _s2_body,
        out_type=jax.ShapeDtypeStruct((NC, NP, H), f32),
        mesh=_mesh,
        compiler_params=_scp,
        scratch_types=[
            pltpu.VMEM((NP,), f32),
            pltpu.VMEM((NPQ,), f32),
            pltpu.VMEM((BE,), i32),
            pltpu.VMEM((BE,), i32),
            pltpu.VMEM((CH,), i32),
            pltpu.VMEM((BE,), f32),
            pltpu.VMEM((BE,), f32),
            pltpu.VMEM((CH, H), f32),
            pltpu.VMEM((4, 16), f32),
            pltpu.VMEM((8, H), f32),
            pltpu.VMEM_SHARED((NP, H), f32),
        ],
    )
    part2 = s2(xp, part1, src_e, dst_e, a_e, q_e, pk)

    BLK = 1280
    h2 = pl.pallas_call(
        _t1_body,
        grid=(NP // BLK,),
        in_specs=[
            pl.BlockSpec((BLK, 1), lambda i: (i, 0)),
            pl.BlockSpec((BLK, 2), lambda i: (i, 0)),
            pl.BlockSpec((2, BLK, H), lambda i: (0, i, 0)),
            pl.BlockSpec((8, H), lambda i: (0, 0)),
            pl.BlockSpec((4, H), lambda i: (0, 0)),
            pl.BlockSpec((H, H), lambda i: (0, 0)),
        ],
        out_specs=pl.BlockSpec((BLK, H), lambda i: (i, 0)),
        out_shape=jax.ShapeDtypeStruct((NP, H), f32),
    )(xp.reshape(NP, 1), part1.T, part2, pk, bt, nn2_W)

    s3 = pl.kernel(
        _s3_body,
        out_type=jax.ShapeDtypeStruct((NC, NP, H), f32),
        mesh=_mesh,
        compiler_params=_scp,
        scratch_types=[
            pltpu.VMEM((BE,), i32),
            pltpu.VMEM((BE,), i32),
            pltpu.VMEM((CH,), i32),
            pltpu.VMEM((CH,), i32),
            pltpu.VMEM((BE,), f32),
            pltpu.VMEM((BE,), f32),
            pltpu.VMEM((CH, H), f32),
            pltpu.VMEM((4, 16), f32),
            pltpu.VMEM((8, H), f32),
            pltpu.VMEM_SHARED((NP, H), f32),
            pltpu.SemaphoreType.DMA,
        ],
    )
    part3 = s3(h2, src_e, dst_e, a_e, q_e, pk)

    A, B, AB = pl.pallas_call(
        _t2_body,
        grid=(NP // BLK,),
        in_specs=[
            pl.BlockSpec((BLK, H), lambda i: (i, 0)),
            pl.BlockSpec((2, BLK, H), lambda i: (0, i, 0)),
            pl.BlockSpec((4, H), lambda i: (0, 0)),
            pl.BlockSpec((H, H), lambda i: (0, 0)),
            pl.BlockSpec((H, DD), lambda i: (0, 0)),
            pl.BlockSpec((H, DD), lambda i: (0, 0)),
            pl.BlockSpec((1, DD), lambda i: (0, 0)),
            pl.BlockSpec((H, 2), lambda i: (0, 0)),
            pl.BlockSpec((1, 2), lambda i: (0, 0)),
        ],
        out_specs=(pl.BlockSpec((BLK, DD), lambda i: (i, 0)),
                   pl.BlockSpec((BLK, DD), lambda i: (i, 0)),
                   pl.BlockSpec((BLK, 2), lambda i: (i, 0))),
        out_shape=(jax.ShapeDtypeStruct((NP, DD), f32),
                   jax.ShapeDtypeStruct((NP, DD), f32),
                   jax.ShapeDtypeStruct((NP, 2), f32)),
    )(h2, part3, bt, nn3_W, wa_m, wb_m, ba_m, wc, bc)

    s4 = pl.kernel(
        _s4_body,
        out_type=jax.ShapeDtypeStruct((E,), f32),
        mesh=_mesh,
        compiler_params=_scp,
        scratch_types=[
            pltpu.VMEM((EW,), i32),
            pltpu.VMEM((EW,), i32),
            pltpu.VMEM((CH,), i32),
            pltpu.VMEM((CH,), i32),
            pltpu.VMEM((EW,), f32),
            pltpu.VMEM((NP,), f32),
            pltpu.VMEM((NP,), f32),
            pltpu.VMEM((CH, DD), f32),
            pltpu.VMEM((CH, DD), f32),
            pltpu.VMEM((EW,), f32),
            pltpu.VMEM((4, 16), f32),
            pltpu.VMEM((2, DD), f32),
            pltpu.VMEM((4, 16), f32),
            pltpu.SemaphoreType.DMA,
            pltpu.SemaphoreType.DMA,
        ],
    )
    out = s4(A, B, AB.T, src_e, dst_e, q_e, cd, sc4)
    return out.reshape(E, 1)


# speedup vs baseline: 7.4899x; 1.0827x over previous
import functools
import math

import jax
import jax.numpy as jnp
from jax import lax
from jax.experimental import pallas as pl
from jax.experimental.pallas import tpu as pltpu
from jax.experimental.pallas import tpu_sc as plsc

N = 10000
NP = 10240
E = 320000
H = 128
DD = 256
NC = 2
NS = 16
NW = NC * NS
L = 16
EW = E // NW
CH = 80
NCH = EW // CH
NG = CH // L
RT = NP // NS
NB = 5
BCH = NCH // NB
BE = BCH * CH
NPQ = NP // 4

_mesh = plsc.VectorSubcoreMesh(core_axis_name="c", subcore_axis_name="s",
                               num_cores=NC, num_subcores=NS)
_scp = pltpu.CompilerParams(needs_layout_passes=False)

f32 = jnp.float32
i32 = jnp.int32


def _prep_body(t_ref, ns_ref, em_W, em_b, emo_W, emo_b,
               lin1_W, lin1_b, lin2_W, lin2_b, lin3_W, lin3_b,
               nn1_W, nn1_b, nn2_b, nn3_b,
               tm1_W, tm1_b, tm2_W, tm2_b, dec2_b,
               pk_ref, bt_ref, s16_ref):
    t = t_ref[...]
    ns = ns_ref[...]
    tt = t / ns * ns * 4.0
    idx = lax.broadcasted_iota(i32, (1, 64), 1).astype(f32)
    emb = jnp.exp(idx * (-(math.log(10000.0) / 63.0)))
    emb = tt * emb
    te0 = jnp.concatenate([jnp.sin(emb), jnp.cos(emb)], axis=-1)
    h = jnp.maximum(te0 @ tm1_W[...] + tm1_b[...], 0.0)
    te = h @ tm2_W[...] + tm2_b[...]

    def uvc(lw, lb):
        u = em_W[...] @ lw[:64]
        v = emo_W[...] @ lw[64:]
        c = em_b[...] @ lw[:64] + emo_b[...] @ lw[64:] + lb[...]
        return u, v, c

    u1, v1, c1 = uvc(lin1_W, lin1_b)
    u2, v2, c2 = uvc(lin2_W, lin2_b)
    u3, v3, c3 = uvc(lin3_W, lin3_b)

    b1r = nn1_b[...] + te
    pk_ref[0:1, :] = nn1_W[...]
    pk_ref[1:2, :] = u2
    pk_ref[2:3, :] = v2
    pk_ref[3:4, :] = b1r + c2
    pk_ref[4:5, :] = u3
    pk_ref[5:6, :] = v3
    pk_ref[6:7, :] = c3
    pk_ref[7:8, :] = jnp.zeros((1, H), f32)

    bt_ref[0:1, :] = b1r
    bt_ref[1:2, :] = nn2_b[...] + te
    bt_ref[2:3, :] = nn3_b[...] + te
    bt_ref[3:4, :] = jnp.zeros((1, H), f32)

    ones = jnp.ones((1, 16), f32)
    s16_ref[0:1, :] = u1 * ones
    s16_ref[1:2, :] = v1 * ones
    s16_ref[2:3, :] = c1 * ones
    s16_ref[3:4, :] = dec2_b[...] * ones
    s16_ref[4:8, :] = jnp.zeros((4, 16), f32)


def _s1_body(x_hbm, src_hbm, dst_hbm, a_hbm, q_hbm, s16_hbm,
             out_hbm,
             xbuf, srcp, dstp, abuf, qbuf, aggr, s16v, sumb, res, stage):
    c = lax.axis_index("c")
    s = lax.axis_index("s")
    wid = s * NC + c
    eb = wid * EW
    pltpu.sync_copy(x_hbm, xbuf)
    pltpu.sync_copy(src_hbm.at[pl.ds(eb, EW)], srcp)
    pltpu.sync_copy(dst_hbm.at[pl.ds(eb, EW)], dstp)
    pltpu.sync_copy(a_hbm.at[pl.ds(eb, EW)], abuf)
    pltpu.sync_copy(q_hbm.at[pl.ds(eb, EW)], qbuf)
    pltpu.sync_copy(s16_hbm, s16v)

    zero = jnp.zeros((L,), f32)

    def zbody(i, carry):
        aggr[pl.ds(i * L, L)] = zero
        return carry
    lax.fori_loop(0, NP // L, zbody, 0)

    u1 = s16v[0, :]
    v1 = s16v[1, :]
    c1 = s16v[2, :]

    def grp(g, carry):
        sl = pl.ds(g * L, L)
        src16 = srcp[sl]
        dst16 = dstp[sl]
        x16 = plsc.load_gather(xbuf, [src16])
        a16 = abuf[sl]
        q16 = qbuf[sl]
        m = jnp.maximum(x16 + a16 * u1 + q16 * v1 + c1, 0.0)
        plsc.addupdate_scatter(aggr, [dst16], m)
        return carry
    lax.fori_loop(0, EW // L, grp, 0)

    pltpu.sync_copy(aggr, stage.at[s])
    plsc.subcore_barrier()
    cs = s * RT
    pltpu.sync_copy(stage.at[:, pl.ds(cs, RT)], sumb)

    def rbody(i, carry):
        acc = sumb[0, pl.ds(i * L, L)]
        for r in range(1, NS):
            acc = acc + sumb[r, pl.ds(i * L, L)]
        res[pl.ds(i * L, L)] = acc
        return carry
    lax.fori_loop(0, RT // L, rbody, 0)
    pltpu.sync_copy(res, out_hbm.at[c, pl.ds(cs, RT)])


def _s2_body(x_hbm, p1_hbm, src_hbm, dst_hbm, a_hbm, q_hbm, pk_hbm,
             out_hbm,
             sbuf, tmp, srcp, dstp, dstc2, abuf, qbuf, msg2, t16, pkv, aggr,
             ssem0, ssem1):
    c = lax.axis_index("c")
    s = lax.axis_index("s")
    wid = s * NC + c
    eb = wid * EW
    pltpu.sync_copy(x_hbm, sbuf)
    for p in range(2):
        for qtr in range(4):
            pltpu.sync_copy(p1_hbm.at[p, pl.ds(qtr * NPQ, NPQ)], tmp)

            def addb(i, carry, _q=qtr):
                sbuf[pl.ds(_q * NPQ + i * L, L)] = (
                    sbuf[pl.ds(_q * NPQ + i * L, L)] + tmp[pl.ds(i * L, L)])
                return carry
            lax.fori_loop(0, NPQ // L, addb, 0)

    pltpu.sync_copy(pk_hbm, pkv)

    zero = jnp.zeros((L,), f32)

    def zb(i, carry):
        for j in range(H // L):
            msg2[i, pl.ds(j * L, L)] = zero
        return carry
    lax.fori_loop(0, CH, zb, 0)
    rstart = s * RT
    for b in range(RT // CH):
        pltpu.sync_copy(msg2.at[pl.ds(0, CH)],
                        aggr.at[pl.ds(rstart + b * CH, CH)])
    plsc.subcore_barrier()

    w = [pkv[0, pl.ds(j * L, L)] for j in range(H // L)]
    u = [pkv[1, pl.ds(j * L, L)] for j in range(H // L)]
    v = [pkv[2, pl.ds(j * L, L)] for j in range(H // L)]
    bb = [pkv[3, pl.ds(j * L, L)] for j in range(H // L)]

    for blk in range(NB):
        pltpu.sync_copy(src_hbm.at[pl.ds(eb + blk * BE, BE)], srcp)
        pltpu.sync_copy(dst_hbm.at[pl.ds(eb + blk * BE, BE)], dstp)
        pltpu.sync_copy(a_hbm.at[pl.ds(eb + blk * BE, BE)], abuf)
        pltpu.sync_copy(q_hbm.at[pl.ds(eb + blk * BE, BE)], qbuf)

        def chunk(k, carry, _blk=blk):
            kg = _blk * BCH + k
            p = lax.rem(kg, 2)
            base = p * CH

            @pl.when(jnp.logical_and(kg >= 2, p == 0))
            def _w0():
                pltpu.make_async_copy(msg2.at[pl.ds(0, CH)],
                                      aggr.at[dstc2.at[0]], ssem0).wait()

            @pl.when(jnp.logical_and(kg >= 2, p == 1))
            def _w1():
                pltpu.make_async_copy(msg2.at[pl.ds(CH, CH)],
                                      aggr.at[dstc2.at[1]], ssem1).wait()

            def grp(g, carry2):
                sl = pl.ds(k * CH + g * L, L)
                src16 = srcp[sl]
                dstc2[p, pl.ds(g * L, L)] = dstp[sl]
                s16 = plsc.load_gather(sbuf, [src16])
                t16[0, :] = s16
                t16[1, :] = abuf[sl]
                t16[2, :] = qbuf[sl]
                for j in range(L):
                    jidx = jnp.full((L,), j, i32)
                    sv = plsc.load_gather(t16.at[0], [jidx])
                    av = plsc.load_gather(t16.at[1], [jidx])
                    qv = plsc.load_gather(t16.at[2], [jidx])
                    for cg in range(H // L):
                        tt = sv * w[cg] + av * u[cg] + qv * v[cg] + bb[cg]
                        msg2[base + g * L + j, pl.ds(cg * L, L)] = (
                            jnp.maximum(tt, 0.0))
                return carry2
            lax.fori_loop(0, NG, grp, 0)

            @pl.when(p == 0)
            def _s0():
                pltpu.async_copy(msg2.at[pl.ds(0, CH)],
                                 aggr.at[dstc2.at[0]], ssem0, add=True)

            @pl.when(p == 1)
            def _s1():
                pltpu.async_copy(msg2.at[pl.ds(CH, CH)],
                                 aggr.at[dstc2.at[1]], ssem1, add=True)
            return carry
        lax.fori_loop(0, BCH, chunk, 0)

    pltpu.make_async_copy(msg2.at[pl.ds(0, CH)],
                          aggr.at[dstc2.at[0]], ssem0).wait()
    pltpu.make_async_copy(msg2.at[pl.ds(CH, CH)],
                          aggr.at[dstc2.at[1]], ssem1).wait()
    plsc.subcore_barrier()
    pltpu.sync_copy(aggr.at[pl.ds(rstart, RT)],
                    out_hbm.at[c, pl.ds(rstart, RT)])


def _s3_body(h2_hbm, src_hbm, dst_hbm, a_hbm, q_hbm, pk_hbm,
             out_hbm,
             srcp, dstp, srcc2, dstc2, abuf, qbuf, rows2, t16, pkv, aggr,
             gsem0, gsem1, ssem0, ssem1):
    c = lax.axis_index("c")
    s = lax.axis_index("s")
    wid = s * NC + c
    eb = wid * EW
    pltpu.sync_copy(pk_hbm, pkv)
    pltpu.sync_copy(src_hbm.at[pl.ds(eb, EW)], srcp)
    pltpu.sync_copy(dst_hbm.at[pl.ds(eb, EW)], dstp)

    zero = jnp.zeros((L,), f32)

    def zb(i, carry):
        for j in range(H // L):
            rows2[i, pl.ds(j * L, L)] = zero
        return carry
    lax.fori_loop(0, CH, zb, 0)
    rstart = s * RT
    for b in range(RT // CH):
        pltpu.sync_copy(rows2.at[pl.ds(0, CH)],
                        aggr.at[pl.ds(rstart + b * CH, CH)])
    plsc.subcore_barrier()

    u = [pkv[4, pl.ds(j * L, L)] for j in range(H // L)]
    v = [pkv[5, pl.ds(j * L, L)] for j in range(H // L)]
    cc = [pkv[6, pl.ds(j * L, L)] for j in range(H // L)]

    for g in range(NG):
        srcc2[0, pl.ds(g * L, L)] = srcp[pl.ds(g * L, L)]
        dstc2[0, pl.ds(g * L, L)] = dstp[pl.ds(g * L, L)]
    pltpu.async_copy(h2_hbm.at[srcc2.at[0]], rows2.at[pl.ds(0, CH)], gsem0)

    for blk in range(NB):
        pltpu.sync_copy(a_hbm.at[pl.ds(eb + blk * BE, BE)], abuf)
        pltpu.sync_copy(q_hbm.at[pl.ds(eb + blk * BE, BE)], qbuf)

        def chunk(k, carry, _blk=blk):
            kg = _blk * BCH + k
            p = lax.rem(kg, 2)
            base = p * CH

            @pl.when(p == 0)
            def _wg0():
                pltpu.make_async_copy(h2_hbm.at[srcc2.at[0]],
                                      rows2.at[pl.ds(0, CH)], gsem0).wait()

            @pl.when(p == 1)
            def _wg1():
                pltpu.make_async_copy(h2_hbm.at[srcc2.at[1]],
                                      rows2.at[pl.ds(CH, CH)], gsem1).wait()

            def grp(g, carry2):
                sl = pl.ds(k * CH + g * L, L)
                t16[1, :] = abuf[sl]
                t16[2, :] = qbuf[sl]
                for j in range(L):
                    jidx = jnp.full((L,), j, i32)
                    av = plsc.load_gather(t16.at[1], [jidx])
                    qv = plsc.load_gather(t16.at[2], [jidx])
                    e = base + g * L + j
                    for cg in range(H // L):
                        slc = pl.ds(cg * L, L)
                        tt = rows2[e, slc] + av * u[cg] + qv * v[cg] + cc[cg]
                        rows2[e, slc] = jnp.maximum(tt, 0.0)
                return carry2
            lax.fori_loop(0, NG, grp, 0)

            @pl.when(p == 0)
            def _ss0():
                pltpu.async_copy(rows2.at[pl.ds(0, CH)],
                                 aggr.at[dstc2.at[0]], ssem0, add=True)

            @pl.when(p == 1)
            def _ss1():
                pltpu.async_copy(rows2.at[pl.ds(CH, CH)],
                                 aggr.at[dstc2.at[1]], ssem1, add=True)

            @pl.when(jnp.logical_and(kg < NCH - 1, p == 1))
            def _pf0():
                pltpu.make_async_copy(rows2.at[pl.ds(0, CH)],
                                      aggr.at[dstc2.at[0]], ssem0).wait()

                def cpy(g, carry2):
                    srcc2[0, pl.ds(g * L, L)] = (
                        srcp[pl.ds((kg + 1) * CH + g * L, L)])
                    dstc2[0, pl.ds(g * L, L)] = (
                        dstp[pl.ds((kg + 1) * CH + g * L, L)])
                    return carry2
                lax.fori_loop(0, NG, cpy, 0)
                pltpu.async_copy(h2_hbm.at[srcc2.at[0]],
                                 rows2.at[pl.ds(0, CH)], gsem0)

            @pl.when(jnp.logical_and(kg < NCH - 1,
                                     jnp.logical_and(p == 0, kg >= 1)))
            def _pf1():
                pltpu.make_async_copy(rows2.at[pl.ds(CH, CH)],
                                      aggr.at[dstc2.at[1]], ssem1).wait()

                def cpy(g, carry2):
                    srcc2[1, pl.ds(g * L, L)] = (
                        srcp[pl.ds((kg + 1) * CH + g * L, L)])
                    dstc2[1, pl.ds(g * L, L)] = (
                        dstp[pl.ds((kg + 1) * CH + g * L, L)])
                    return carry2
                lax.fori_loop(0, NG, cpy, 0)
                pltpu.async_copy(h2_hbm.at[srcc2.at[1]],
                                 rows2.at[pl.ds(CH, CH)], gsem1)

            @pl.when(jnp.logical_and(kg == 0, True))
            def _pf1a():
                def cpy(g, carry2):
                    srcc2[1, pl.ds(g * L, L)] = srcp[pl.ds(CH + g * L, L)]
                    dstc2[1, pl.ds(g * L, L)] = dstp[pl.ds(CH + g * L, L)]
                    return carry2
                lax.fori_loop(0, NG, cpy, 0)
                pltpu.async_copy(h2_hbm.at[srcc2.at[1]],
                                 rows2.at[pl.ds(CH, CH)], gsem1)
            return carry
        lax.fori_loop(0, BCH, chunk, 0)

    pltpu.make_async_copy(rows2.at[pl.ds(0, CH)],
                          aggr.at[dstc2.at[0]], ssem0).wait()
    pltpu.make_async_copy(rows2.at[pl.ds(CH, CH)],
                          aggr.at[dstc2.at[1]], ssem1).wait()
    plsc.subcore_barrier()
    pltpu.sync_copy(aggr.at[pl.ds(rstart, RT)],
                    out_hbm.at[c, pl.ds(rstart, RT)])


def _s4_body(A_hbm, B_hbm, ab_hbm, src_hbm, dst_hbm, q_hbm, cd_hbm, sc4_hbm,
             out_hbm,
             srcp, dstp, srcc, dstc, qbuf, a256b, b256b, rowsA, rowsB, outb,
             t16, cdv, sc4v, semA, semB):
    c = lax.axis_index("c")
    s = lax.axis_index("s")
    wid = s * NC + c
    eb = wid * EW
    pltpu.sync_copy(src_hbm.at[pl.ds(eb, EW)], srcp)
    pltpu.sync_copy(dst_hbm.at[pl.ds(eb, EW)], dstp)
    pltpu.sync_copy(q_hbm.at[pl.ds(eb, EW)], qbuf)
    pltpu.sync_copy(ab_hbm.at[0], a256b)
    pltpu.sync_copy(ab_hbm.at[1], b256b)
    pltpu.sync_copy(cd_hbm, cdv)
    pltpu.sync_copy(sc4_hbm, sc4v)

    wq = [cdv[0, pl.ds(j * L, L)] for j in range(DD // L)]
    dv = [cdv[1, pl.ds(j * L, L)] for j in range(DD // L)]
    wq256 = sc4v[0, :]
    d256 = sc4v[1, :]
    db2 = sc4v[2, :]
    lane15 = lax.broadcasted_iota(i32, (L,), 0) == (L - 1)

    def chunk(k, carry):
        def cpy(g, carry2):
            srcc[pl.ds(g * L, L)] = srcp[pl.ds(k * CH + g * L, L)]
            dstc[pl.ds(g * L, L)] = dstp[pl.ds(k * CH + g * L, L)]
            return carry2
        lax.fori_loop(0, NG, cpy, 0)
        cpA = pltpu.async_copy(A_hbm.at[srcc], rowsA, semA)
        cpB = pltpu.async_copy(B_hbm.at[dstc], rowsB, semB)
        cpA.wait()
        cpB.wait()

        def grp(g, carry2):
            sl16 = pl.ds(k * CH + g * L, L)
            q16 = qbuf[sl16]
            src16 = srcp[sl16]
            dst16 = dstp[sl16]
            a256 = plsc.load_gather(a256b, [src16])
            b256 = plsc.load_gather(b256b, [dst16])
            v256 = jnp.maximum(a256 + b256 + q16 * wq256, 0.0) * d256
            t16[0, :] = q16
            t16[3, :] = v256
            for j in range(L):
                jidx = jnp.full((L,), j, i32)
                qv = plsc.load_gather(t16.at[0], [jidx])
                vv = plsc.load_gather(t16.at[3], [jidx])
                e = g * L + j
                acc = jnp.zeros((L,), f32)
                for cg in range(DD // L):
                    sl = pl.ds(cg * L, L)
                    tt = jnp.maximum(rowsA[e, sl] + rowsB[e, sl]
                                     + qv * wq[cg], 0.0)
                    acc = acc + tt * dv[cg]
                tot = plsc.cumsum(acc) + (db2 + vv)
                eidx = jnp.full((L,), k * CH, i32) + (g * L + j)
                plsc.store_scatter(outb, [eidx], tot, mask=lane15)
            return carry2
        return lax.fori_loop(0, NG, grp, carry)
    lax.fori_loop(0, NCH, chunk, 0)
    pltpu.sync_copy(outb, out_hbm.at[pl.ds(eb, EW)])


def _t1_body(xc_ref, p1_ref, p2_ref, pk_ref, bt_ref, nn2_ref, h2_ref):
    s_col = xc_ref[...] + p1_ref[:, 0:1] + p1_ref[:, 1:2]
    h1 = s_col * pk_ref[0:1, :] + bt_ref[0:1, :]
    h2in = h1 + p2_ref[0] + p2_ref[1]
    h2_ref[...] = h2in @ nn2_ref[...] + bt_ref[1:2, :]


def _t2_body(h2_ref, p3_ref, bt_ref, nn3_ref, wa_ref, wb_ref, ba_ref,
             wc_ref, bc_ref, a_ref, b_ref, ab_ref):
    h3 = (h2_ref[...] + p3_ref[0] + p3_ref[1]) @ nn3_ref[...] + bt_ref[2:3, :]
    a_ref[...] = h3 @ wa_ref[...] + ba_ref[...]
    b_ref[...] = h3 @ wb_ref[...]
    ab_ref[...] = h3 @ wc_ref[...] + bc_ref[...]


def kernel(x, edge_attr, q_Y_sample, adj, t, num_steps, batch,
           em_W, em_b, emo_W, emo_b,
           lin1_W, lin1_b, nn1_W, nn1_b,
           lin2_W, lin2_b, nn2_W, nn2_b,
           lin3_W, lin3_b, nn3_W, nn3_b,
           dec1_W, dec1_b, dec2_W, dec2_b,
           tm1_W, tm1_b, tm2_W, tm2_b):
    xp = jnp.pad(x[:, 0], (0, NP - N))
    a_e = edge_attr[:, 0]
    q_e = q_Y_sample[:, 0]
    src_e = adj[0]
    dst_e = adj[1]
    t2 = t.reshape(1, 1)
    ns2 = jnp.asarray(num_steps, f32).reshape(1, 1)
    lin1b2 = lin1_b.reshape(1, 1)
    wa_m = dec1_W[:128, :DD]
    wb_m = dec1_W[128:256, :DD]
    ba_m = dec1_b[:DD].reshape(1, DD)
    wc = jnp.stack([dec1_W[:128, DD], dec1_W[128:256, DD]], axis=1)
    bc = jnp.stack([dec1_b[DD], jnp.zeros((), f32)]).reshape(1, 2)
    cd = jnp.stack([dec1_W[256, :DD], dec2_W[:DD, 0]])
    sc4 = jnp.stack([jnp.full((16,), dec1_W[256, DD], f32),
                     jnp.full((16,), dec2_W[DD, 0], f32),
                     jnp.full((16,), dec2_b[0], f32),
                     jnp.zeros((16,), f32)])

    pk, bt, s16 = pl.pallas_call(
        _prep_body,
        out_shape=(jax.ShapeDtypeStruct((8, H), f32),
                   jax.ShapeDtypeStruct((4, H), f32),
                   jax.ShapeDtypeStruct((8, 16), f32)),
    )(t2, ns2, em_W, em_b.reshape(1, 64), emo_W, emo_b.reshape(1, 64),
      lin1_W, lin1b2, lin2_W, lin2_b.reshape(1, H),
      lin3_W, lin3_b.reshape(1, H),
      nn1_W, nn1_b.reshape(1, H), nn2_b.reshape(1, H), nn3_b.reshape(1, H),
      tm1_W, tm1_b.reshape(1, 256), tm2_W, tm2_b.reshape(1, H),
      dec2_b.reshape(1, 1))

    s1 = pl.kernel(
        _s1_body,
        out_type=jax.ShapeDtypeStruct((NC, NP), f32),
        mesh=_mesh,
        compiler_params=_scp,
        scratch_types=[
            pltpu.VMEM((NP,), f32),
            pltpu.VMEM((EW,), i32),
            pltpu.VMEM((EW,), i32),
            pltpu.VMEM((EW,), f32),
            pltpu.VMEM((EW,), f32),
            pltpu.VMEM((NP,), f32),
            pltpu.VMEM((8, 16), f32),
            pltpu.VMEM((NS, RT), f32),
            pltpu.VMEM((RT,), f32),
            pltpu.VMEM_SHARED((NS, NP), f32),
        ],
    )
    part1 = s1(xp, src_e, dst_e, a_e, q_e, s16)

    s2 = pl.kernel(
        _s2_body,
        out_type=jax.ShapeDtypeStruct((NC, NP, H), f32),
        mesh=_mesh,
        compiler_params=_scp,
        scratch_types=[
            pltpu.VMEM((NP,), f32),
            pltpu.VMEM((NPQ,), f32),
            pltpu.VMEM((BE,), i32),
            pltpu.VMEM((BE,), i32),
            pltpu.VMEM((2, CH), i32),
            pltpu.VMEM((BE,), f32),
            pltpu.VMEM((BE,), f32),
            pltpu.VMEM((2 * CH, H), f32),
            pltpu.VMEM((4, 16), f32),
            pltpu.VMEM((8, H), f32),
            pltpu.VMEM_SHARED((NP, H), f32),
            pltpu.SemaphoreType.DMA,
            pltpu.SemaphoreType.DMA,
        ],
    )
    part2 = s2(xp, part1, src_e, dst_e, a_e, q_e, pk)

    BLK = 1280
    h2 = pl.pallas_call(
        _t1_body,
        grid=(NP // BLK,),
        in_specs=[
            pl.BlockSpec((BLK, 1), lambda i: (i, 0)),
            pl.BlockSpec((BLK, 2), lambda i: (i, 0)),
            pl.BlockSpec((2, BLK, H), lambda i: (0, i, 0)),
            pl.BlockSpec((8, H), lambda i: (0, 0)),
            pl.BlockSpec((4, H), lambda i: (0, 0)),
            pl.BlockSpec((H, H), lambda i: (0, 0)),
        ],
        out_specs=pl.BlockSpec((BLK, H), lambda i: (i, 0)),
        out_shape=jax.ShapeDtypeStruct((NP, H), f32),
    )(xp.reshape(NP, 1), part1.T, part2, pk, bt, nn2_W)

    s3 = pl.kernel(
        _s3_body,
        out_type=jax.ShapeDtypeStruct((NC, NP, H), f32),
        mesh=_mesh,
        compiler_params=_scp,
        scratch_types=[
            pltpu.VMEM((EW,), i32),
            pltpu.VMEM((EW,), i32),
            pltpu.VMEM((2, CH), i32),
            pltpu.VMEM((2, CH), i32),
            pltpu.VMEM((BE,), f32),
            pltpu.VMEM((BE,), f32),
            pltpu.VMEM((2 * CH, H), f32),
            pltpu.VMEM((4, 16), f32),
            pltpu.VMEM((8, H), f32),
            pltpu.VMEM_SHARED((NP, H), f32),
            pltpu.SemaphoreType.DMA,
            pltpu.SemaphoreType.DMA,
            pltpu.SemaphoreType.DMA,
            pltpu.SemaphoreType.DMA,
        ],
    )
    part3 = s3(h2, src_e, dst_e, a_e, q_e, pk)

    A, B, AB = pl.pallas_call(
        _t2_body,
        grid=(NP // BLK,),
        in_specs=[
            pl.BlockSpec((BLK, H), lambda i: (i, 0)),
            pl.BlockSpec((2, BLK, H), lambda i: (0, i, 0)),
            pl.BlockSpec((4, H), lambda i: (0, 0)),
            pl.BlockSpec((H, H), lambda i: (0, 0)),
            pl.BlockSpec((H, DD), lambda i: (0, 0)),
            pl.BlockSpec((H, DD), lambda i: (0, 0)),
            pl.BlockSpec((1, DD), lambda i: (0, 0)),
            pl.BlockSpec((H, 2), lambda i: (0, 0)),
            pl.BlockSpec((1, 2), lambda i: (0, 0)),
        ],
        out_specs=(pl.BlockSpec((BLK, DD), lambda i: (i, 0)),
                   pl.BlockSpec((BLK, DD), lambda i: (i, 0)),
                   pl.BlockSpec((BLK, 2), lambda i: (i, 0))),
        out_shape=(jax.ShapeDtypeStruct((NP, DD), f32),
                   jax.ShapeDtypeStruct((NP, DD), f32),
                   jax.ShapeDtypeStruct((NP, 2), f32)),
    )(h2, part3, bt, nn3_W, wa_m, wb_m, ba_m, wc, bc)

    s4 = pl.kernel(
        _s4_body,
        out_type=jax.ShapeDtypeStruct((E,), f32),
        mesh=_mesh,
        compiler_params=_scp,
        scratch_types=[
            pltpu.VMEM((EW,), i32),
            pltpu.VMEM((EW,), i32),
            pltpu.VMEM((CH,), i32),
            pltpu.VMEM((CH,), i32),
            pltpu.VMEM((EW,), f32),
            pltpu.VMEM((NP,), f32),
            pltpu.VMEM((NP,), f32),
            pltpu.VMEM((CH, DD), f32),
            pltpu.VMEM((CH, DD), f32),
            pltpu.VMEM((EW,), f32),
            pltpu.VMEM((4, 16), f32),
            pltpu.VMEM((2, DD), f32),
            pltpu.VMEM((4, 16), f32),
            pltpu.SemaphoreType.DMA,
            pltpu.SemaphoreType.DMA,
        ],
    )
    out = s4(A, B, AB.T, src_e, dst_e, q_e, cd, sc4)
    return out.reshape(E, 1)


# speedup vs baseline: 9.1157x; 1.2171x over previous
import functools
import math

import jax
import jax.numpy as jnp
from jax import lax
from jax.experimental import pallas as pl
from jax.experimental.pallas import tpu as pltpu
from jax.experimental.pallas import tpu_sc as plsc

N = 10000
NP = 10240
E = 320000
H = 128
DD = 256
NC = 2
NS = 16
NW = NC * NS
L = 16
EW = E // NW
CH = 80
NCH = EW // CH
NG = CH // L
RT = NP // NS
NB = 5
BCH = NCH // NB
BE = BCH * CH
NPQ = NP // 4

_mesh = plsc.VectorSubcoreMesh(core_axis_name="c", subcore_axis_name="s",
                               num_cores=NC, num_subcores=NS)
_scp = pltpu.CompilerParams(needs_layout_passes=False)

f32 = jnp.float32
i32 = jnp.int32


def _prep_body(t_ref, ns_ref, em_W, em_b, emo_W, emo_b,
               lin1_W, lin1_b, lin2_W, lin2_b, lin3_W, lin3_b,
               nn1_W, nn1_b, nn2_b, nn3_b,
               tm1_W, tm1_b, tm2_W, tm2_b, dec2_b,
               pk_ref, bt_ref, s16_ref):
    t = t_ref[...]
    ns = ns_ref[...]
    tt = t / ns * ns * 4.0
    idx = lax.broadcasted_iota(i32, (1, 64), 1).astype(f32)
    emb = jnp.exp(idx * (-(math.log(10000.0) / 63.0)))
    emb = tt * emb
    te0 = jnp.concatenate([jnp.sin(emb), jnp.cos(emb)], axis=-1)
    h = jnp.maximum(te0 @ tm1_W[...] + tm1_b[...], 0.0)
    te = h @ tm2_W[...] + tm2_b[...]

    def uvc(lw, lb):
        u = em_W[...] @ lw[:64]
        v = emo_W[...] @ lw[64:]
        c = em_b[...] @ lw[:64] + emo_b[...] @ lw[64:] + lb[...]
        return u, v, c

    u1, v1, c1 = uvc(lin1_W, lin1_b)
    u2, v2, c2 = uvc(lin2_W, lin2_b)
    u3, v3, c3 = uvc(lin3_W, lin3_b)

    b1r = nn1_b[...] + te
    pk_ref[0:1, :] = nn1_W[...]
    pk_ref[1:2, :] = u2
    pk_ref[2:3, :] = v2
    pk_ref[3:4, :] = b1r + c2
    pk_ref[4:5, :] = u3
    pk_ref[5:6, :] = v3
    pk_ref[6:7, :] = c3
    pk_ref[7:8, :] = jnp.zeros((1, H), f32)

    bt_ref[0:1, :] = b1r
    bt_ref[1:2, :] = nn2_b[...] + te
    bt_ref[2:3, :] = nn3_b[...] + te
    bt_ref[3:4, :] = jnp.zeros((1, H), f32)

    ones = jnp.ones((1, 16), f32)
    s16_ref[0:1, :] = u1 * ones
    s16_ref[1:2, :] = v1 * ones
    s16_ref[2:3, :] = c1 * ones
    s16_ref[3:4, :] = dec2_b[...] * ones
    s16_ref[4:8, :] = jnp.zeros((4, 16), f32)


def _s1_body(x_hbm, src_hbm, dst_hbm, a_hbm, q_hbm, s16_hbm,
             out_hbm,
             xbuf, srcp, dstp, abuf, qbuf, aggr, s16v, sumb, res, stage):
    c = lax.axis_index("c")
    s = lax.axis_index("s")
    wid = s * NC + c
    eb = wid * EW
    pltpu.sync_copy(x_hbm, xbuf)
    pltpu.sync_copy(src_hbm.at[pl.ds(eb, EW)], srcp)
    pltpu.sync_copy(dst_hbm.at[pl.ds(eb, EW)], dstp)
    pltpu.sync_copy(a_hbm.at[pl.ds(eb, EW)], abuf)
    pltpu.sync_copy(q_hbm.at[pl.ds(eb, EW)], qbuf)
    pltpu.sync_copy(s16_hbm, s16v)

    zero = jnp.zeros((L,), f32)

    def zbody(i, carry):
        aggr[pl.ds(i * L, L)] = zero
        return carry
    lax.fori_loop(0, NP // L, zbody, 0)

    u1 = s16v[0, :]
    v1 = s16v[1, :]
    c1 = s16v[2, :]

    def grp(g, carry):
        sl = pl.ds(g * L, L)
        src16 = srcp[sl]
        dst16 = dstp[sl]
        x16 = plsc.load_gather(xbuf, [src16])
        a16 = abuf[sl]
        q16 = qbuf[sl]
        m = jnp.maximum(x16 + a16 * u1 + q16 * v1 + c1, 0.0)
        plsc.addupdate_scatter(aggr, [dst16], m)
        return carry
    lax.fori_loop(0, EW // L, grp, 0)

    pltpu.sync_copy(aggr, stage.at[s])
    plsc.subcore_barrier()
    cs = s * RT
    pltpu.sync_copy(stage.at[:, pl.ds(cs, RT)], sumb)

    def rbody(i, carry):
        acc = sumb[0, pl.ds(i * L, L)]
        for r in range(1, NS):
            acc = acc + sumb[r, pl.ds(i * L, L)]
        res[pl.ds(i * L, L)] = acc
        return carry
    lax.fori_loop(0, RT // L, rbody, 0)
    pltpu.sync_copy(res, out_hbm.at[c, pl.ds(cs, RT)])


def _s2_body(x_hbm, p1_hbm, src_hbm, dst_hbm, a_hbm, q_hbm, pk_hbm,
             out_hbm,
             sbuf, tmp, srcp, dstp, dstc2, abuf, qbuf, msg2, t16, pkv, aggr,
             ssem0, ssem1):
    c = lax.axis_index("c")
    s = lax.axis_index("s")
    wid = s * NC + c
    eb = wid * EW
    pltpu.sync_copy(x_hbm, sbuf)
    for p in range(2):
        for qtr in range(4):
            pltpu.sync_copy(p1_hbm.at[p, pl.ds(qtr * NPQ, NPQ)], tmp)

            def addb(i, carry, _q=qtr):
                sbuf[pl.ds(_q * NPQ + i * L, L)] = (
                    sbuf[pl.ds(_q * NPQ + i * L, L)] + tmp[pl.ds(i * L, L)])
                return carry
            lax.fori_loop(0, NPQ // L, addb, 0)

    pltpu.sync_copy(pk_hbm, pkv)

    zero = jnp.zeros((L,), f32)

    def zb(i, carry):
        for j in range(H // L):
            msg2[i, pl.ds(j * L, L)] = zero
        return carry
    lax.fori_loop(0, CH, zb, 0)
    rstart = s * RT
    for b in range(RT // CH):
        pltpu.sync_copy(msg2.at[pl.ds(0, CH)],
                        aggr.at[pl.ds(rstart + b * CH, CH)])
    plsc.subcore_barrier()

    w = [pkv[0, pl.ds(j * L, L)] for j in range(H // L)]
    u = [pkv[1, pl.ds(j * L, L)] for j in range(H // L)]
    v = [pkv[2, pl.ds(j * L, L)] for j in range(H // L)]
    bb = [pkv[3, pl.ds(j * L, L)] for j in range(H // L)]

    for blk in range(NB):
        pltpu.sync_copy(src_hbm.at[pl.ds(eb + blk * BE, BE)], srcp)
        pltpu.sync_copy(dst_hbm.at[pl.ds(eb + blk * BE, BE)], dstp)
        pltpu.sync_copy(a_hbm.at[pl.ds(eb + blk * BE, BE)], abuf)
        pltpu.sync_copy(q_hbm.at[pl.ds(eb + blk * BE, BE)], qbuf)

        def chunk(k, carry, _blk=blk):
            kg = _blk * BCH + k
            p = lax.rem(kg, 2)
            base = p * CH

            @pl.when(jnp.logical_and(kg >= 2, p == 0))
            def _w0():
                pltpu.make_async_copy(msg2.at[pl.ds(0, CH)],
                                      aggr.at[dstc2.at[0]], ssem0).wait()

            @pl.when(jnp.logical_and(kg >= 2, p == 1))
            def _w1():
                pltpu.make_async_copy(msg2.at[pl.ds(CH, CH)],
                                      aggr.at[dstc2.at[1]], ssem1).wait()

            def grp(g, carry2):
                sl = pl.ds(k * CH + g * L, L)
                src16 = srcp[sl]
                dstc2[p, pl.ds(g * L, L)] = dstp[sl]
                s16 = plsc.load_gather(sbuf, [src16])
                t16[0, :] = s16
                t16[1, :] = abuf[sl]
                t16[2, :] = qbuf[sl]
                for j in range(L):
                    jidx = jnp.full((L,), j, i32)
                    sv = plsc.load_gather(t16.at[0], [jidx])
                    av = plsc.load_gather(t16.at[1], [jidx])
                    qv = plsc.load_gather(t16.at[2], [jidx])
                    for cg in range(H // L):
                        tt = sv * w[cg] + av * u[cg] + qv * v[cg] + bb[cg]
                        msg2[base + g * L + j, pl.ds(cg * L, L)] = (
                            jnp.maximum(tt, 0.0))
                return carry2
            lax.fori_loop(0, NG, grp, 0)

            @pl.when(p == 0)
            def _s0():
                pltpu.async_copy(msg2.at[pl.ds(0, CH)],
                                 aggr.at[dstc2.at[0]], ssem0, add=True)

            @pl.when(p == 1)
            def _s1():
                pltpu.async_copy(msg2.at[pl.ds(CH, CH)],
                                 aggr.at[dstc2.at[1]], ssem1, add=True)
            return carry
        lax.fori_loop(0, BCH, chunk, 0)

    pltpu.make_async_copy(msg2.at[pl.ds(0, CH)],
                          aggr.at[dstc2.at[0]], ssem0).wait()
    pltpu.make_async_copy(msg2.at[pl.ds(CH, CH)],
                          aggr.at[dstc2.at[1]], ssem1).wait()
    plsc.subcore_barrier()
    pltpu.sync_copy(aggr.at[pl.ds(rstart, RT)],
                    out_hbm.at[c, pl.ds(rstart, RT)])


def _s3_body(h2_hbm, src_hbm, dst_hbm, a_hbm, q_hbm, pk_hbm,
             out_hbm,
             srcp, dstp, srcc2, dstc2, abuf, qbuf, rows2, t16, pkv, aggr,
             gsem0, gsem1, ssem0, ssem1):
    c = lax.axis_index("c")
    s = lax.axis_index("s")
    wid = s * NC + c
    eb = wid * EW
    pltpu.sync_copy(pk_hbm, pkv)
    pltpu.sync_copy(src_hbm.at[pl.ds(eb, EW)], srcp)
    pltpu.sync_copy(dst_hbm.at[pl.ds(eb, EW)], dstp)

    zero = jnp.zeros((L,), f32)

    def zb(i, carry):
        for j in range(H // L):
            rows2[i, pl.ds(j * L, L)] = zero
        return carry
    lax.fori_loop(0, CH, zb, 0)
    rstart = s * RT
    for b in range(RT // CH):
        pltpu.sync_copy(rows2.at[pl.ds(0, CH)],
                        aggr.at[pl.ds(rstart + b * CH, CH)])
    plsc.subcore_barrier()

    u = [pkv[4, pl.ds(j * L, L)] for j in range(H // L)]
    v = [pkv[5, pl.ds(j * L, L)] for j in range(H // L)]
    cc = [pkv[6, pl.ds(j * L, L)] for j in range(H // L)]

    for g in range(NG):
        srcc2[0, pl.ds(g * L, L)] = srcp[pl.ds(g * L, L)]
        dstc2[0, pl.ds(g * L, L)] = dstp[pl.ds(g * L, L)]
    pltpu.async_copy(h2_hbm.at[srcc2.at[0]], rows2.at[pl.ds(0, CH)], gsem0)

    for blk in range(NB):
        pltpu.sync_copy(a_hbm.at[pl.ds(eb + blk * BE, BE)], abuf)
        pltpu.sync_copy(q_hbm.at[pl.ds(eb + blk * BE, BE)], qbuf)

        def chunk(k, carry, _blk=blk):
            kg = _blk * BCH + k
            p = lax.rem(kg, 2)
            base = p * CH

            @pl.when(p == 0)
            def _wg0():
                pltpu.make_async_copy(h2_hbm.at[srcc2.at[0]],
                                      rows2.at[pl.ds(0, CH)], gsem0).wait()

            @pl.when(p == 1)
            def _wg1():
                pltpu.make_async_copy(h2_hbm.at[srcc2.at[1]],
                                      rows2.at[pl.ds(CH, CH)], gsem1).wait()

            def grp(g, carry2):
                sl = pl.ds(k * CH + g * L, L)
                t16[1, :] = abuf[sl]
                t16[2, :] = qbuf[sl]
                for j in range(L):
                    jidx = jnp.full((L,), j, i32)
                    av = plsc.load_gather(t16.at[1], [jidx])
                    qv = plsc.load_gather(t16.at[2], [jidx])
                    e = base + g * L + j
                    for cg in range(H // L):
                        slc = pl.ds(cg * L, L)
                        tt = rows2[e, slc] + av * u[cg] + qv * v[cg] + cc[cg]
                        rows2[e, slc] = jnp.maximum(tt, 0.0)
                return carry2
            lax.fori_loop(0, NG, grp, 0)

            @pl.when(p == 0)
            def _ss0():
                pltpu.async_copy(rows2.at[pl.ds(0, CH)],
                                 aggr.at[dstc2.at[0]], ssem0, add=True)

            @pl.when(p == 1)
            def _ss1():
                pltpu.async_copy(rows2.at[pl.ds(CH, CH)],
                                 aggr.at[dstc2.at[1]], ssem1, add=True)

            @pl.when(jnp.logical_and(kg < NCH - 1, p == 1))
            def _pf0():
                pltpu.make_async_copy(rows2.at[pl.ds(0, CH)],
                                      aggr.at[dstc2.at[0]], ssem0).wait()

                def cpy(g, carry2):
                    srcc2[0, pl.ds(g * L, L)] = (
                        srcp[pl.ds((kg + 1) * CH + g * L, L)])
                    dstc2[0, pl.ds(g * L, L)] = (
                        dstp[pl.ds((kg + 1) * CH + g * L, L)])
                    return carry2
                lax.fori_loop(0, NG, cpy, 0)
                pltpu.async_copy(h2_hbm.at[srcc2.at[0]],
                                 rows2.at[pl.ds(0, CH)], gsem0)

            @pl.when(jnp.logical_and(kg < NCH - 1,
                                     jnp.logical_and(p == 0, kg >= 1)))
            def _pf1():
                pltpu.make_async_copy(rows2.at[pl.ds(CH, CH)],
                                      aggr.at[dstc2.at[1]], ssem1).wait()

                def cpy(g, carry2):
                    srcc2[1, pl.ds(g * L, L)] = (
                        srcp[pl.ds((kg + 1) * CH + g * L, L)])
                    dstc2[1, pl.ds(g * L, L)] = (
                        dstp[pl.ds((kg + 1) * CH + g * L, L)])
                    return carry2
                lax.fori_loop(0, NG, cpy, 0)
                pltpu.async_copy(h2_hbm.at[srcc2.at[1]],
                                 rows2.at[pl.ds(CH, CH)], gsem1)

            @pl.when(jnp.logical_and(kg == 0, True))
            def _pf1a():
                def cpy(g, carry2):
                    srcc2[1, pl.ds(g * L, L)] = srcp[pl.ds(CH + g * L, L)]
                    dstc2[1, pl.ds(g * L, L)] = dstp[pl.ds(CH + g * L, L)]
                    return carry2
                lax.fori_loop(0, NG, cpy, 0)
                pltpu.async_copy(h2_hbm.at[srcc2.at[1]],
                                 rows2.at[pl.ds(CH, CH)], gsem1)
            return carry
        lax.fori_loop(0, BCH, chunk, 0)

    pltpu.make_async_copy(rows2.at[pl.ds(0, CH)],
                          aggr.at[dstc2.at[0]], ssem0).wait()
    pltpu.make_async_copy(rows2.at[pl.ds(CH, CH)],
                          aggr.at[dstc2.at[1]], ssem1).wait()
    plsc.subcore_barrier()
    pltpu.sync_copy(aggr.at[pl.ds(rstart, RT)],
                    out_hbm.at[c, pl.ds(rstart, RT)])


def _s4_body(A_hbm, B_hbm, ab_hbm, src_hbm, dst_hbm, q_hbm, cd_hbm, sc4_hbm,
             out_hbm,
             srcp, dstp, srcc2, dstc2, qbuf, a256b, b256b, rows2A, rows2B,
             outb, t16, cdv, sc4v, gsA0, gsA1, gsB0, gsB1):
    c = lax.axis_index("c")
    s = lax.axis_index("s")
    wid = s * NC + c
    eb = wid * EW
    pltpu.sync_copy(ab_hbm.at[0], a256b)
    pltpu.sync_copy(ab_hbm.at[1], b256b)
    pltpu.sync_copy(cd_hbm, cdv)
    pltpu.sync_copy(sc4_hbm, sc4v)

    wq = [cdv[0, pl.ds(j * L, L)] for j in range(DD // L)]
    dv = [cdv[1, pl.ds(j * L, L)] for j in range(DD // L)]
    wq256 = sc4v[0, :]
    d256 = sc4v[1, :]
    db2 = sc4v[2, :]
    lane15 = lax.broadcasted_iota(i32, (L,), 0) == (L - 1)

    for blk in range(NB):
        pltpu.sync_copy(src_hbm.at[pl.ds(eb + blk * BE, BE)], srcp)
        pltpu.sync_copy(dst_hbm.at[pl.ds(eb + blk * BE, BE)], dstp)
        pltpu.sync_copy(q_hbm.at[pl.ds(eb + blk * BE, BE)], qbuf)

        for g in range(NG):
            srcc2[0, pl.ds(g * L, L)] = srcp[pl.ds(g * L, L)]
            dstc2[0, pl.ds(g * L, L)] = dstp[pl.ds(g * L, L)]
        pltpu.async_copy(A_hbm.at[srcc2.at[0]], rows2A.at[pl.ds(0, CH)], gsA0)
        pltpu.async_copy(B_hbm.at[dstc2.at[0]], rows2B.at[pl.ds(0, CH)], gsB0)

        def chunk(k, carry):
            p = lax.rem(k, 2)
            base = p * CH

            @pl.when(p == 0)
            def _w0():
                pltpu.make_async_copy(A_hbm.at[srcc2.at[0]],
                                      rows2A.at[pl.ds(0, CH)], gsA0).wait()
                pltpu.make_async_copy(B_hbm.at[dstc2.at[0]],
                                      rows2B.at[pl.ds(0, CH)], gsB0).wait()

            @pl.when(p == 1)
            def _w1():
                pltpu.make_async_copy(A_hbm.at[srcc2.at[1]],
                                      rows2A.at[pl.ds(CH, CH)], gsA1).wait()
                pltpu.make_async_copy(B_hbm.at[dstc2.at[1]],
                                      rows2B.at[pl.ds(CH, CH)], gsB1).wait()

            @pl.when(jnp.logical_and(k < BCH - 1, p == 0))
            def _pf1():
                def cpy(g, carry2):
                    srcc2[1, pl.ds(g * L, L)] = (
                        srcp[pl.ds((k + 1) * CH + g * L, L)])
                    dstc2[1, pl.ds(g * L, L)] = (
                        dstp[pl.ds((k + 1) * CH + g * L, L)])
                    return carry2
                lax.fori_loop(0, NG, cpy, 0)
                pltpu.async_copy(A_hbm.at[srcc2.at[1]],
                                 rows2A.at[pl.ds(CH, CH)], gsA1)
                pltpu.async_copy(B_hbm.at[dstc2.at[1]],
                                 rows2B.at[pl.ds(CH, CH)], gsB1)

            @pl.when(jnp.logical_and(k < BCH - 1, p == 1))
            def _pf0():
                def cpy(g, carry2):
                    srcc2[0, pl.ds(g * L, L)] = (
                        srcp[pl.ds((k + 1) * CH + g * L, L)])
                    dstc2[0, pl.ds(g * L, L)] = (
                        dstp[pl.ds((k + 1) * CH + g * L, L)])
                    return carry2
                lax.fori_loop(0, NG, cpy, 0)
                pltpu.async_copy(A_hbm.at[srcc2.at[0]],
                                 rows2A.at[pl.ds(0, CH)], gsA0)
                pltpu.async_copy(B_hbm.at[dstc2.at[0]],
                                 rows2B.at[pl.ds(0, CH)], gsB0)

            def grp(g, carry2):
                sl16 = pl.ds(k * CH + g * L, L)
                q16 = qbuf[sl16]
                src16 = srcp[sl16]
                dst16 = dstp[sl16]
                a256 = plsc.load_gather(a256b, [src16])
                b256 = plsc.load_gather(b256b, [dst16])
                v256 = jnp.maximum(a256 + b256 + q16 * wq256, 0.0) * d256
                t16[0, :] = q16
                t16[3, :] = v256
                for j in range(L):
                    jidx = jnp.full((L,), j, i32)
                    qv = plsc.load_gather(t16.at[0], [jidx])
                    vv = plsc.load_gather(t16.at[3], [jidx])
                    e = base + g * L + j
                    acc = jnp.zeros((L,), f32)
                    for cg in range(DD // L):
                        sl = pl.ds(cg * L, L)
                        tt = jnp.maximum(rows2A[e, sl] + rows2B[e, sl]
                                         + qv * wq[cg], 0.0)
                        acc = acc + tt * dv[cg]
                    tot = plsc.cumsum(acc) + (db2 + vv)
                    eidx = jnp.full((L,), k * CH, i32) + (g * L + j)
                    plsc.store_scatter(outb, [eidx], tot, mask=lane15)
                return carry2
            return lax.fori_loop(0, NG, grp, carry)
        lax.fori_loop(0, BCH, chunk, 0)
        pltpu.sync_copy(outb, out_hbm.at[pl.ds(eb + blk * BE, BE)])


def _t1_body(xc_ref, p1_ref, p2_ref, pk_ref, bt_ref, nn2_ref, h2_ref):
    s_col = xc_ref[...] + p1_ref[:, 0:1] + p1_ref[:, 1:2]
    h1 = s_col * pk_ref[0:1, :] + bt_ref[0:1, :]
    h2in = h1 + p2_ref[0] + p2_ref[1]
    h2_ref[...] = h2in @ nn2_ref[...] + bt_ref[1:2, :]


def _t2_body(h2_ref, p3_ref, bt_ref, nn3_ref, wa_ref, wb_ref, ba_ref,
             wc_ref, bc_ref, a_ref, b_ref, ab_ref):
    h3 = (h2_ref[...] + p3_ref[0] + p3_ref[1]) @ nn3_ref[...] + bt_ref[2:3, :]
    a_ref[...] = h3 @ wa_ref[...] + ba_ref[...]
    b_ref[...] = h3 @ wb_ref[...]
    ab_ref[...] = h3 @ wc_ref[...] + bc_ref[...]


def kernel(x, edge_attr, q_Y_sample, adj, t, num_steps, batch,
           em_W, em_b, emo_W, emo_b,
           lin1_W, lin1_b, nn1_W, nn1_b,
           lin2_W, lin2_b, nn2_W, nn2_b,
           lin3_W, lin3_b, nn3_W, nn3_b,
           dec1_W, dec1_b, dec2_W, dec2_b,
           tm1_W, tm1_b, tm2_W, tm2_b):
    xp = jnp.pad(x[:, 0], (0, NP - N))
    a_e = edge_attr[:, 0]
    q_e = q_Y_sample[:, 0]
    src_e = adj[0]
    dst_e = adj[1]
    t2 = t.reshape(1, 1)
    ns2 = jnp.asarray(num_steps, f32).reshape(1, 1)
    lin1b2 = lin1_b.reshape(1, 1)
    wa_m = dec1_W[:128, :DD]
    wb_m = dec1_W[128:256, :DD]
    ba_m = dec1_b[:DD].reshape(1, DD)
    wc = jnp.stack([dec1_W[:128, DD], dec1_W[128:256, DD]], axis=1)
    bc = jnp.stack([dec1_b[DD], jnp.zeros((), f32)]).reshape(1, 2)
    cd = jnp.stack([dec1_W[256, :DD], dec2_W[:DD, 0]])
    sc4 = jnp.stack([jnp.full((16,), dec1_W[256, DD], f32),
                     jnp.full((16,), dec2_W[DD, 0], f32),
                     jnp.full((16,), dec2_b[0], f32),
                     jnp.zeros((16,), f32)])

    pk, bt, s16 = pl.pallas_call(
        _prep_body,
        out_shape=(jax.ShapeDtypeStruct((8, H), f32),
                   jax.ShapeDtypeStruct((4, H), f32),
                   jax.ShapeDtypeStruct((8, 16), f32)),
    )(t2, ns2, em_W, em_b.reshape(1, 64), emo_W, emo_b.reshape(1, 64),
      lin1_W, lin1b2, lin2_W, lin2_b.reshape(1, H),
      lin3_W, lin3_b.reshape(1, H),
      nn1_W, nn1_b.reshape(1, H), nn2_b.reshape(1, H), nn3_b.reshape(1, H),
      tm1_W, tm1_b.reshape(1, 256), tm2_W, tm2_b.reshape(1, H),
      dec2_b.reshape(1, 1))

    s1 = pl.kernel(
        _s1_body,
        out_type=jax.ShapeDtypeStruct((NC, NP), f32),
        mesh=_mesh,
        compiler_params=_scp,
        scratch_types=[
            pltpu.VMEM((NP,), f32),
            pltpu.VMEM((EW,), i32),
            pltpu.VMEM((EW,), i32),
            pltpu.VMEM((EW,), f32),
            pltpu.VMEM((EW,), f32),
            pltpu.VMEM((NP,), f32),
            pltpu.VMEM((8, 16), f32),
            pltpu.VMEM((NS, RT), f32),
            pltpu.VMEM((RT,), f32),
            pltpu.VMEM_SHARED((NS, NP), f32),
        ],
    )
    part1 = s1(xp, src_e, dst_e, a_e, q_e, s16)

    s2 = pl.kernel(
        _s2_body,
        out_type=jax.ShapeDtypeStruct((NC, NP, H), f32),
        mesh=_mesh,
        compiler_params=_scp,
        scratch_types=[
            pltpu.VMEM((NP,), f32),
            pltpu.VMEM((NPQ,), f32),
            pltpu.VMEM((BE,), i32),
            pltpu.VMEM((BE,), i32),
            pltpu.VMEM((2, CH), i32),
            pltpu.VMEM((BE,), f32),
            pltpu.VMEM((BE,), f32),
            pltpu.VMEM((2 * CH, H), f32),
            pltpu.VMEM((4, 16), f32),
            pltpu.VMEM((8, H), f32),
            pltpu.VMEM_SHARED((NP, H), f32),
            pltpu.SemaphoreType.DMA,
            pltpu.SemaphoreType.DMA,
        ],
    )
    part2 = s2(xp, part1, src_e, dst_e, a_e, q_e, pk)

    BLK = 1280
    h2 = pl.pallas_call(
        _t1_body,
        grid=(NP // BLK,),
        in_specs=[
            pl.BlockSpec((BLK, 1), lambda i: (i, 0)),
            pl.BlockSpec((BLK, 2), lambda i: (i, 0)),
            pl.BlockSpec((2, BLK, H), lambda i: (0, i, 0)),
            pl.BlockSpec((8, H), lambda i: (0, 0)),
            pl.BlockSpec((4, H), lambda i: (0, 0)),
            pl.BlockSpec((H, H), lambda i: (0, 0)),
        ],
        out_specs=pl.BlockSpec((BLK, H), lambda i: (i, 0)),
        out_shape=jax.ShapeDtypeStruct((NP, H), f32),
    )(xp.reshape(NP, 1), part1.T, part2, pk, bt, nn2_W)

    s3 = pl.kernel(
        _s3_body,
        out_type=jax.ShapeDtypeStruct((NC, NP, H), f32),
        mesh=_mesh,
        compiler_params=_scp,
        scratch_types=[
            pltpu.VMEM((EW,), i32),
            pltpu.VMEM((EW,), i32),
            pltpu.VMEM((2, CH), i32),
            pltpu.VMEM((2, CH), i32),
            pltpu.VMEM((BE,), f32),
            pltpu.VMEM((BE,), f32),
            pltpu.VMEM((2 * CH, H), f32),
            pltpu.VMEM((4, 16), f32),
            pltpu.VMEM((8, H), f32),
            pltpu.VMEM_SHARED((NP, H), f32),
            pltpu.SemaphoreType.DMA,
            pltpu.SemaphoreType.DMA,
            pltpu.SemaphoreType.DMA,
            pltpu.SemaphoreType.DMA,
        ],
    )
    part3 = s3(h2, src_e, dst_e, a_e, q_e, pk)

    A, B, AB = pl.pallas_call(
        _t2_body,
        grid=(NP // BLK,),
        in_specs=[
            pl.BlockSpec((BLK, H), lambda i: (i, 0)),
            pl.BlockSpec((2, BLK, H), lambda i: (0, i, 0)),
            pl.BlockSpec((4, H), lambda i: (0, 0)),
            pl.BlockSpec((H, H), lambda i: (0, 0)),
            pl.BlockSpec((H, DD), lambda i: (0, 0)),
            pl.BlockSpec((H, DD), lambda i: (0, 0)),
            pl.BlockSpec((1, DD), lambda i: (0, 0)),
            pl.BlockSpec((H, 2), lambda i: (0, 0)),
            pl.BlockSpec((1, 2), lambda i: (0, 0)),
        ],
        out_specs=(pl.BlockSpec((BLK, DD), lambda i: (i, 0)),
                   pl.BlockSpec((BLK, DD), lambda i: (i, 0)),
                   pl.BlockSpec((BLK, 2), lambda i: (i, 0))),
        out_shape=(jax.ShapeDtypeStruct((NP, DD), f32),
                   jax.ShapeDtypeStruct((NP, DD), f32),
                   jax.ShapeDtypeStruct((NP, 2), f32)),
    )(h2, part3, bt, nn3_W, wa_m, wb_m, ba_m, wc, bc)

    s4 = pl.kernel(
        _s4_body,
        out_type=jax.ShapeDtypeStruct((E,), f32),
        mesh=_mesh,
        compiler_params=_scp,
        scratch_types=[
            pltpu.VMEM((BE,), i32),
            pltpu.VMEM((BE,), i32),
            pltpu.VMEM((2, CH), i32),
            pltpu.VMEM((2, CH), i32),
            pltpu.VMEM((BE,), f32),
            pltpu.VMEM((NP,), f32),
            pltpu.VMEM((NP,), f32),
            pltpu.VMEM((2 * CH, DD), f32),
            pltpu.VMEM((2 * CH, DD), f32),
            pltpu.VMEM((BE,), f32),
            pltpu.VMEM((4, 16), f32),
            pltpu.VMEM((2, DD), f32),
            pltpu.VMEM((4, 16), f32),
            pltpu.SemaphoreType.DMA,
            pltpu.SemaphoreType.DMA,
            pltpu.SemaphoreType.DMA,
            pltpu.SemaphoreType.DMA,
        ],
    )
    out = s4(A, B, AB.T, src_e, dst_e, q_e, cd, sc4)
    return out.reshape(E, 1)


# speedup vs baseline: 10.0602x; 1.1036x over previous
import functools
import math

import jax
import jax.numpy as jnp
from jax import lax
from jax.experimental import pallas as pl
from jax.experimental.pallas import tpu as pltpu
from jax.experimental.pallas import tpu_sc as plsc

N = 10000
NP = 10240
E = 320000
H = 128
DD = 256
NC = 2
NS = 16
NW = NC * NS
L = 16
EW = E // NW
CH = 80
NCH = EW // CH
NG = CH // L
RT = NP // NS
NB = 5
BCH = NCH // NB
BE = BCH * CH
NPQ = NP // 4

_mesh = plsc.VectorSubcoreMesh(core_axis_name="c", subcore_axis_name="s",
                               num_cores=NC, num_subcores=NS)
_scp = pltpu.CompilerParams(needs_layout_passes=False)

f32 = jnp.float32
i32 = jnp.int32


def _prep_body(t_ref, ns_ref, em_W, em_b, emo_W, emo_b,
               lin1_W, lin1_b, lin2_W, lin2_b, lin3_W, lin3_b,
               nn1_W, nn1_b, nn2_b, nn3_b,
               tm1_W, tm1_b, tm2_W, tm2_b, dec2_b,
               pk_ref, bt_ref, s16_ref):
    t = t_ref[...]
    ns = ns_ref[...]
    tt = t / ns * ns * 4.0
    idx = lax.broadcasted_iota(i32, (1, 64), 1).astype(f32)
    emb = jnp.exp(idx * (-(math.log(10000.0) / 63.0)))
    emb = tt * emb
    te0 = jnp.concatenate([jnp.sin(emb), jnp.cos(emb)], axis=-1)
    h = jnp.maximum(te0 @ tm1_W[...] + tm1_b[...], 0.0)
    te = h @ tm2_W[...] + tm2_b[...]

    def uvc(lw, lb):
        u = em_W[...] @ lw[:64]
        v = emo_W[...] @ lw[64:]
        c = em_b[...] @ lw[:64] + emo_b[...] @ lw[64:] + lb[...]
        return u, v, c

    u1, v1, c1 = uvc(lin1_W, lin1_b)
    u2, v2, c2 = uvc(lin2_W, lin2_b)
    u3, v3, c3 = uvc(lin3_W, lin3_b)

    b1r = nn1_b[...] + te
    pk_ref[0:1, :] = nn1_W[...]
    pk_ref[1:2, :] = u2
    pk_ref[2:3, :] = v2
    pk_ref[3:4, :] = b1r + c2
    pk_ref[4:5, :] = u3
    pk_ref[5:6, :] = v3
    pk_ref[6:7, :] = c3
    pk_ref[7:8, :] = jnp.zeros((1, H), f32)

    bt_ref[0:1, :] = b1r
    bt_ref[1:2, :] = nn2_b[...] + te
    bt_ref[2:3, :] = nn3_b[...] + te
    bt_ref[3:4, :] = jnp.zeros((1, H), f32)

    ones = jnp.ones((1, 16), f32)
    s16_ref[0:1, :] = u1 * ones
    s16_ref[1:2, :] = v1 * ones
    s16_ref[2:3, :] = c1 * ones
    s16_ref[3:4, :] = dec2_b[...] * ones
    s16_ref[4:8, :] = jnp.zeros((4, 16), f32)


def _s1_body(x_hbm, src_hbm, dst_hbm, a_hbm, q_hbm, s16_hbm,
             out_hbm,
             xbuf, srcp, dstp, abuf, qbuf, aggr, s16v, sumb, res, stage):
    c = lax.axis_index("c")
    s = lax.axis_index("s")
    wid = s * NC + c
    eb = wid * EW
    pltpu.sync_copy(x_hbm, xbuf)
    pltpu.sync_copy(src_hbm.at[pl.ds(eb, EW)], srcp)
    pltpu.sync_copy(dst_hbm.at[pl.ds(eb, EW)], dstp)
    pltpu.sync_copy(a_hbm.at[pl.ds(eb, EW)], abuf)
    pltpu.sync_copy(q_hbm.at[pl.ds(eb, EW)], qbuf)
    pltpu.sync_copy(s16_hbm, s16v)

    zero = jnp.zeros((L,), f32)

    def zbody(i, carry):
        aggr[pl.ds(i * L, L)] = zero
        return carry
    lax.fori_loop(0, NP // L, zbody, 0)

    u1 = s16v[0, :]
    v1 = s16v[1, :]
    c1 = s16v[2, :]

    def grp(g, carry):
        sl = pl.ds(g * L, L)
        src16 = srcp[sl]
        dst16 = dstp[sl]
        x16 = plsc.load_gather(xbuf, [src16])
        a16 = abuf[sl]
        q16 = qbuf[sl]
        m = jnp.maximum(x16 + a16 * u1 + q16 * v1 + c1, 0.0)
        plsc.addupdate_scatter(aggr, [dst16], m)
        return carry
    lax.fori_loop(0, EW // L, grp, 0)

    pltpu.sync_copy(aggr, stage.at[s])
    plsc.subcore_barrier()
    cs = s * RT
    pltpu.sync_copy(stage.at[:, pl.ds(cs, RT)], sumb)

    def rbody(i, carry):
        acc = sumb[0, pl.ds(i * L, L)]
        for r in range(1, NS):
            acc = acc + sumb[r, pl.ds(i * L, L)]
        res[pl.ds(i * L, L)] = acc
        return carry
    lax.fori_loop(0, RT // L, rbody, 0)
    pltpu.sync_copy(res, out_hbm.at[c, pl.ds(cs, RT)])


def _s2_body(x_hbm, p1_hbm, src_hbm, dst_hbm, a_hbm, q_hbm, pk_hbm,
             out_hbm,
             sbuf, tmp, srcp, dstp, dstc2, abuf, qbuf, msg2, t16, pkv, aggr,
             ssem0, ssem1):
    c = lax.axis_index("c")
    s = lax.axis_index("s")
    wid = s * NC + c
    eb = wid * EW
    pltpu.sync_copy(x_hbm, sbuf)
    for p in range(2):
        for qtr in range(4):
            pltpu.sync_copy(p1_hbm.at[p, pl.ds(qtr * NPQ, NPQ)], tmp)

            def addb(i, carry, _q=qtr):
                sbuf[pl.ds(_q * NPQ + i * L, L)] = (
                    sbuf[pl.ds(_q * NPQ + i * L, L)] + tmp[pl.ds(i * L, L)])
                return carry
            lax.fori_loop(0, NPQ // L, addb, 0)

    pltpu.sync_copy(pk_hbm, pkv)

    zero = jnp.zeros((L,), f32)

    def zb(i, carry):
        for j in range(H // L):
            msg2[i, pl.ds(j * L, L)] = zero
        return carry
    lax.fori_loop(0, CH, zb, 0)
    rstart = s * RT
    for b in range(RT // CH):
        pltpu.sync_copy(msg2.at[pl.ds(0, CH)],
                        aggr.at[pl.ds(rstart + b * CH, CH)])
    plsc.subcore_barrier()

    w = [pkv[0, pl.ds(j * L, L)] for j in range(H // L)]
    u = [pkv[1, pl.ds(j * L, L)] for j in range(H // L)]
    v = [pkv[2, pl.ds(j * L, L)] for j in range(H // L)]
    bb = [pkv[3, pl.ds(j * L, L)] for j in range(H // L)]

    for blk in range(NB):
        pltpu.sync_copy(src_hbm.at[pl.ds(eb + blk * BE, BE)], srcp)
        pltpu.sync_copy(dst_hbm.at[pl.ds(eb + blk * BE, BE)], dstp)
        pltpu.sync_copy(a_hbm.at[pl.ds(eb + blk * BE, BE)], abuf)
        pltpu.sync_copy(q_hbm.at[pl.ds(eb + blk * BE, BE)], qbuf)

        def chunk(k, carry, _blk=blk):
            kg = _blk * BCH + k
            p = lax.rem(kg, 2)
            base = p * CH

            @pl.when(jnp.logical_and(kg >= 2, p == 0))
            def _w0():
                pltpu.make_async_copy(msg2.at[pl.ds(0, CH)],
                                      aggr.at[dstc2.at[0]], ssem0).wait()

            @pl.when(jnp.logical_and(kg >= 2, p == 1))
            def _w1():
                pltpu.make_async_copy(msg2.at[pl.ds(CH, CH)],
                                      aggr.at[dstc2.at[1]], ssem1).wait()

            def grp(g, carry2):
                sl = pl.ds(k * CH + g * L, L)
                src16 = srcp[sl]
                dstc2[p, pl.ds(g * L, L)] = dstp[sl]
                s16 = plsc.load_gather(sbuf, [src16])
                t16[0, :] = s16
                t16[1, :] = abuf[sl]
                t16[2, :] = qbuf[sl]
                for j in range(L):
                    jidx = jnp.full((L,), j, i32)
                    sv = plsc.load_gather(t16.at[0], [jidx])
                    av = plsc.load_gather(t16.at[1], [jidx])
                    qv = plsc.load_gather(t16.at[2], [jidx])
                    for cg in range(H // L):
                        tt = sv * w[cg] + av * u[cg] + qv * v[cg] + bb[cg]
                        msg2[base + g * L + j, pl.ds(cg * L, L)] = (
                            jnp.maximum(tt, 0.0))
                return carry2
            lax.fori_loop(0, NG, grp, 0)

            @pl.when(p == 0)
            def _s0():
                pltpu.async_copy(msg2.at[pl.ds(0, CH)],
                                 aggr.at[dstc2.at[0]], ssem0, add=True)

            @pl.when(p == 1)
            def _s1():
                pltpu.async_copy(msg2.at[pl.ds(CH, CH)],
                                 aggr.at[dstc2.at[1]], ssem1, add=True)
            return carry
        lax.fori_loop(0, BCH, chunk, 0)

    pltpu.make_async_copy(msg2.at[pl.ds(0, CH)],
                          aggr.at[dstc2.at[0]], ssem0).wait()
    pltpu.make_async_copy(msg2.at[pl.ds(CH, CH)],
                          aggr.at[dstc2.at[1]], ssem1).wait()
    plsc.subcore_barrier()
    pltpu.sync_copy(aggr.at[pl.ds(rstart, RT)],
                    out_hbm.at[c, pl.ds(rstart, RT)])


def _s3_body(h2_hbm, src_hbm, dst_hbm, a_hbm, q_hbm, pk_hbm,
             out_hbm,
             srcp, dstp, srcc3, dstc3, abuf, qbuf, rows3, t16, pkv, aggr,
             gs0, gs1, gs2, ss0, ss1, ss2):
    c = lax.axis_index("c")
    s = lax.axis_index("s")
    wid = s * NC + c
    eb = wid * EW
    pltpu.sync_copy(pk_hbm, pkv)

    zero = jnp.zeros((L,), f32)

    def zb(i, carry):
        for j in range(H // L):
            rows3[i, pl.ds(j * L, L)] = zero
        return carry
    lax.fori_loop(0, CH, zb, 0)
    rstart = s * RT
    for b in range(RT // CH):
        pltpu.sync_copy(rows3.at[pl.ds(0, CH)],
                        aggr.at[pl.ds(rstart + b * CH, CH)])
    plsc.subcore_barrier()

    u = [pkv[4, pl.ds(j * L, L)] for j in range(H // L)]
    v = [pkv[5, pl.ds(j * L, L)] for j in range(H // L)]
    cc = [pkv[6, pl.ds(j * L, L)] for j in range(H // L)]

    gsems = (gs0, gs1, gs2)
    ssems = (ss0, ss1, ss2)

    def _gather(buf, sem):
        return pltpu.async_copy(h2_hbm.at[srcc3.at[buf]],
                                rows3.at[pl.ds(buf * CH, CH)], sem)

    def _gwait(buf, sem):
        pltpu.make_async_copy(h2_hbm.at[srcc3.at[buf]],
                              rows3.at[pl.ds(buf * CH, CH)], sem).wait()

    def _scat(buf, sem):
        return pltpu.async_copy(rows3.at[pl.ds(buf * CH, CH)],
                                aggr.at[dstc3.at[buf]], sem, add=True)

    def _swait(buf, sem):
        pltpu.make_async_copy(rows3.at[pl.ds(buf * CH, CH)],
                              aggr.at[dstc3.at[buf]], sem).wait()

    for blk in range(NB):
        pltpu.sync_copy(src_hbm.at[pl.ds(eb + blk * BE, BE)], srcp)
        pltpu.sync_copy(dst_hbm.at[pl.ds(eb + blk * BE, BE)], dstp)
        pltpu.sync_copy(a_hbm.at[pl.ds(eb + blk * BE, BE)], abuf)
        pltpu.sync_copy(q_hbm.at[pl.ds(eb + blk * BE, BE)], qbuf)

        for i in range(2):
            for g in range(NG):
                srcc3[i, pl.ds(g * L, L)] = srcp[pl.ds(i * CH + g * L, L)]
                dstc3[i, pl.ds(g * L, L)] = dstp[pl.ds(i * CH + g * L, L)]
            _gather(i, gsems[i])

        def chunk(k, carry):
            b = lax.rem(k, 3)
            base = b * CH

            for bi in range(3):
                @pl.when(b == bi)
                def _wg(_bi=bi):
                    _gwait(_bi, gsems[_bi])

            def grp(g, carry2):
                sl = pl.ds(k * CH + g * L, L)
                t16[1, :] = abuf[sl]
                t16[2, :] = qbuf[sl]
                for j in range(L):
                    jidx = jnp.full((L,), j, i32)
                    av = plsc.load_gather(t16.at[1], [jidx])
                    qv = plsc.load_gather(t16.at[2], [jidx])
                    e = base + g * L + j
                    for cg in range(H // L):
                        slc = pl.ds(cg * L, L)
                        tt = rows3[e, slc] + av * u[cg] + qv * v[cg] + cc[cg]
                        rows3[e, slc] = jnp.maximum(tt, 0.0)
                return carry2
            lax.fori_loop(0, NG, grp, 0)

            for bi in range(3):
                @pl.when(b == bi)
                def _sc(_bi=bi):
                    _scat(_bi, ssems[_bi])

            bp = lax.rem(k + 2, 3)
            for bi in range(3):
                @pl.when(jnp.logical_and(k < BCH - 2, bp == bi))
                def _pf(_bi=bi):
                    @pl.when(k >= 1)
                    def _wsc():
                        _swait(_bi, ssems[_bi])

                    def cpy(g, carry2):
                        srcc3[_bi, pl.ds(g * L, L)] = (
                            srcp[pl.ds((k + 2) * CH + g * L, L)])
                        dstc3[_bi, pl.ds(g * L, L)] = (
                            dstp[pl.ds((k + 2) * CH + g * L, L)])
                        return carry2
                    lax.fori_loop(0, NG, cpy, 0)
                    _gather(_bi, gsems[_bi])
            return carry
        lax.fori_loop(0, BCH, chunk, 0)

        for bi in range(3):
            _swait(bi, ssems[bi])

    plsc.subcore_barrier()
    pltpu.sync_copy(aggr.at[pl.ds(rstart, RT)],
                    out_hbm.at[c, pl.ds(rstart, RT)])


def _s4_body(A_hbm, B_hbm, ab_hbm, src_hbm, dst_hbm, q_hbm, cd_hbm, sc4_hbm,
             out_hbm,
             srcp, dstp, srcc2, dstc2, qbuf, a256b, b256b, rows2A, rows2B,
             outb, t16, cdv, sc4v, gsA0, gsA1, gsB0, gsB1):
    c = lax.axis_index("c")
    s = lax.axis_index("s")
    wid = s * NC + c
    eb = wid * EW
    pltpu.sync_copy(ab_hbm.at[0], a256b)
    pltpu.sync_copy(ab_hbm.at[1], b256b)
    pltpu.sync_copy(cd_hbm, cdv)
    pltpu.sync_copy(sc4_hbm, sc4v)

    wq = [cdv[0, pl.ds(j * L, L)] for j in range(DD // L)]
    dv = [cdv[1, pl.ds(j * L, L)] for j in range(DD // L)]
    wq256 = sc4v[0, :]
    d256 = sc4v[1, :]
    db2 = sc4v[2, :]
    lane15 = lax.broadcasted_iota(i32, (L,), 0) == (L - 1)

    for blk in range(NB):
        pltpu.sync_copy(src_hbm.at[pl.ds(eb + blk * BE, BE)], srcp)
        pltpu.sync_copy(dst_hbm.at[pl.ds(eb + blk * BE, BE)], dstp)
        pltpu.sync_copy(q_hbm.at[pl.ds(eb + blk * BE, BE)], qbuf)

        for g in range(NG):
            srcc2[0, pl.ds(g * L, L)] = srcp[pl.ds(g * L, L)]
            dstc2[0, pl.ds(g * L, L)] = dstp[pl.ds(g * L, L)]
        pltpu.async_copy(A_hbm.at[srcc2.at[0]], rows2A.at[pl.ds(0, CH)], gsA0)
        pltpu.async_copy(B_hbm.at[dstc2.at[0]], rows2B.at[pl.ds(0, CH)], gsB0)

        def chunk(k, carry):
            p = lax.rem(k, 2)
            base = p * CH

            @pl.when(p == 0)
            def _w0():
                pltpu.make_async_copy(A_hbm.at[srcc2.at[0]],
                                      rows2A.at[pl.ds(0, CH)], gsA0).wait()
                pltpu.make_async_copy(B_hbm.at[dstc2.at[0]],
                                      rows2B.at[pl.ds(0, CH)], gsB0).wait()

            @pl.when(p == 1)
            def _w1():
                pltpu.make_async_copy(A_hbm.at[srcc2.at[1]],
                                      rows2A.at[pl.ds(CH, CH)], gsA1).wait()
                pltpu.make_async_copy(B_hbm.at[dstc2.at[1]],
                                      rows2B.at[pl.ds(CH, CH)], gsB1).wait()

            @pl.when(jnp.logical_and(k < BCH - 1, p == 0))
            def _pf1():
                def cpy(g, carry2):
                    srcc2[1, pl.ds(g * L, L)] = (
                        srcp[pl.ds((k + 1) * CH + g * L, L)])
                    dstc2[1, pl.ds(g * L, L)] = (
                        dstp[pl.ds((k + 1) * CH + g * L, L)])
                    return carry2
                lax.fori_loop(0, NG, cpy, 0)
                pltpu.async_copy(A_hbm.at[srcc2.at[1]],
                                 rows2A.at[pl.ds(CH, CH)], gsA1)
                pltpu.async_copy(B_hbm.at[dstc2.at[1]],
                                 rows2B.at[pl.ds(CH, CH)], gsB1)

            @pl.when(jnp.logical_and(k < BCH - 1, p == 1))
            def _pf0():
                def cpy(g, carry2):
                    srcc2[0, pl.ds(g * L, L)] = (
                        srcp[pl.ds((k + 1) * CH + g * L, L)])
                    dstc2[0, pl.ds(g * L, L)] = (
                        dstp[pl.ds((k + 1) * CH + g * L, L)])
                    return carry2
                lax.fori_loop(0, NG, cpy, 0)
                pltpu.async_copy(A_hbm.at[srcc2.at[0]],
                                 rows2A.at[pl.ds(0, CH)], gsA0)
                pltpu.async_copy(B_hbm.at[dstc2.at[0]],
                                 rows2B.at[pl.ds(0, CH)], gsB0)

            def grp(g, carry2):
                sl16 = pl.ds(k * CH + g * L, L)
                q16 = qbuf[sl16]
                src16 = srcp[sl16]
                dst16 = dstp[sl16]
                a256 = plsc.load_gather(a256b, [src16])
                b256 = plsc.load_gather(b256b, [dst16])
                v256 = jnp.maximum(a256 + b256 + q16 * wq256, 0.0) * d256
                t16[0, :] = q16
                t16[3, :] = v256
                for j in range(L):
                    jidx = jnp.full((L,), j, i32)
                    qv = plsc.load_gather(t16.at[0], [jidx])
                    vv = plsc.load_gather(t16.at[3], [jidx])
                    e = base + g * L + j
                    acc = jnp.zeros((L,), f32)
                    for cg in range(DD // L):
                        sl = pl.ds(cg * L, L)
                        tt = jnp.maximum(rows2A[e, sl] + rows2B[e, sl]
                                         + qv * wq[cg], 0.0)
                        acc = acc + tt * dv[cg]
                    tot = plsc.cumsum(acc) + (db2 + vv)
                    eidx = jnp.full((L,), k * CH, i32) + (g * L + j)
                    plsc.store_scatter(outb, [eidx], tot, mask=lane15)
                return carry2
            return lax.fori_loop(0, NG, grp, carry)
        lax.fori_loop(0, BCH, chunk, 0)
        pltpu.sync_copy(outb, out_hbm.at[pl.ds(eb + blk * BE, BE)])


def _t1_body(xc_ref, p1_ref, p2_ref, pk_ref, bt_ref, nn2_ref, h2_ref):
    s_col = xc_ref[...] + p1_ref[:, 0:1] + p1_ref[:, 1:2]
    h1 = s_col * pk_ref[0:1, :] + bt_ref[0:1, :]
    h2in = h1 + p2_ref[0] + p2_ref[1]
    h2_ref[...] = h2in @ nn2_ref[...] + bt_ref[1:2, :]


def _t2_body(h2_ref, p3_ref, bt_ref, nn3_ref, wa_ref, wb_ref, ba_ref,
             wc_ref, bc_ref, a_ref, b_ref, ab_ref):
    h3 = (h2_ref[...] + p3_ref[0] + p3_ref[1]) @ nn3_ref[...] + bt_ref[2:3, :]
    a_ref[...] = h3 @ wa_ref[...] + ba_ref[...]
    b_ref[...] = h3 @ wb_ref[...]
    ab_ref[...] = h3 @ wc_ref[...] + bc_ref[...]


def kernel(x, edge_attr, q_Y_sample, adj, t, num_steps, batch,
           em_W, em_b, emo_W, emo_b,
           lin1_W, lin1_b, nn1_W, nn1_b,
           lin2_W, lin2_b, nn2_W, nn2_b,
           lin3_W, lin3_b, nn3_W, nn3_b,
           dec1_W, dec1_b, dec2_W, dec2_b,
           tm1_W, tm1_b, tm2_W, tm2_b):
    xp = jnp.pad(x[:, 0], (0, NP - N))
    a_e = edge_attr[:, 0]
    q_e = q_Y_sample[:, 0]
    src_e = adj[0]
    dst_e = adj[1]
    t2 = t.reshape(1, 1)
    ns2 = jnp.asarray(num_steps, f32).reshape(1, 1)
    lin1b2 = lin1_b.reshape(1, 1)
    wa_m = dec1_W[:128, :DD]
    wb_m = dec1_W[128:256, :DD]
    ba_m = dec1_b[:DD].reshape(1, DD)
    wc = jnp.stack([dec1_W[:128, DD], dec1_W[128:256, DD]], axis=1)
    bc = jnp.stack([dec1_b[DD], jnp.zeros((), f32)]).reshape(1, 2)
    cd = jnp.stack([dec1_W[256, :DD], dec2_W[:DD, 0]])
    sc4 = jnp.stack([jnp.full((16,), dec1_W[256, DD], f32),
                     jnp.full((16,), dec2_W[DD, 0], f32),
                     jnp.full((16,), dec2_b[0], f32),
                     jnp.zeros((16,), f32)])

    pk, bt, s16 = pl.pallas_call(
        _prep_body,
        out_shape=(jax.ShapeDtypeStruct((8, H), f32),
                   jax.ShapeDtypeStruct((4, H), f32),
                   jax.ShapeDtypeStruct((8, 16), f32)),
    )(t2, ns2, em_W, em_b.reshape(1, 64), emo_W, emo_b.reshape(1, 64),
      lin1_W, lin1b2, lin2_W, lin2_b.reshape(1, H),
      lin3_W, lin3_b.reshape(1, H),
      nn1_W, nn1_b.reshape(1, H), nn2_b.reshape(1, H), nn3_b.reshape(1, H),
      tm1_W, tm1_b.reshape(1, 256), tm2_W, tm2_b.reshape(1, H),
      dec2_b.reshape(1, 1))

    s1 = pl.kernel(
        _s1_body,
        out_type=jax.ShapeDtypeStruct((NC, NP), f32),
        mesh=_mesh,
        compiler_params=_scp,
        scratch_types=[
            pltpu.VMEM((NP,), f32),
            pltpu.VMEM((EW,), i32),
            pltpu.VMEM((EW,), i32),
            pltpu.VMEM((EW,), f32),
            pltpu.VMEM((EW,), f32),
            pltpu.VMEM((NP,), f32),
            pltpu.VMEM((8, 16), f32),
            pltpu.VMEM((NS, RT), f32),
            pltpu.VMEM((RT,), f32),
            pltpu.VMEM_SHARED((NS, NP), f32),
        ],
    )
    part1 = s1(xp, src_e, dst_e, a_e, q_e, s16)

    s2 = pl.kernel(
        _s2_body,
        out_type=jax.ShapeDtypeStruct((NC, NP, H), f32),
        mesh=_mesh,
        compiler_params=_scp,
        scratch_types=[
            pltpu.VMEM((NP,), f32),
            pltpu.VMEM((NPQ,), f32),
            pltpu.VMEM((BE,), i32),
            pltpu.VMEM((BE,), i32),
            pltpu.VMEM((2, CH), i32),
            pltpu.VMEM((BE,), f32),
            pltpu.VMEM((BE,), f32),
            pltpu.VMEM((2 * CH, H), f32),
            pltpu.VMEM((4, 16), f32),
            pltpu.VMEM((8, H), f32),
            pltpu.VMEM_SHARED((NP, H), f32),
            pltpu.SemaphoreType.DMA,
            pltpu.SemaphoreType.DMA,
        ],
    )
    part2 = s2(xp, part1, src_e, dst_e, a_e, q_e, pk)

    BLK = 1280
    h2 = pl.pallas_call(
        _t1_body,
        grid=(NP // BLK,),
        in_specs=[
            pl.BlockSpec((BLK, 1), lambda i: (i, 0)),
            pl.BlockSpec((BLK, 2), lambda i: (i, 0)),
            pl.BlockSpec((2, BLK, H), lambda i: (0, i, 0)),
            pl.BlockSpec((8, H), lambda i: (0, 0)),
            pl.BlockSpec((4, H), lambda i: (0, 0)),
            pl.BlockSpec((H, H), lambda i: (0, 0)),
        ],
        out_specs=pl.BlockSpec((BLK, H), lambda i: (i, 0)),
        out_shape=jax.ShapeDtypeStruct((NP, H), f32),
    )(xp.reshape(NP, 1), part1.T, part2, pk, bt, nn2_W)

    s3 = pl.kernel(
        _s3_body,
        out_type=jax.ShapeDtypeStruct((NC, NP, H), f32),
        mesh=_mesh,
        compiler_params=_scp,
        scratch_types=[
            pltpu.VMEM((BE,), i32),
            pltpu.VMEM((BE,), i32),
            pltpu.VMEM((3, CH), i32),
            pltpu.VMEM((3, CH), i32),
            pltpu.VMEM((BE,), f32),
            pltpu.VMEM((BE,), f32),
            pltpu.VMEM((3 * CH, H), f32),
            pltpu.VMEM((4, 16), f32),
            pltpu.VMEM((8, H), f32),
            pltpu.VMEM_SHARED((NP, H), f32),
            pltpu.SemaphoreType.DMA,
            pltpu.SemaphoreType.DMA,
            pltpu.SemaphoreType.DMA,
            pltpu.SemaphoreType.DMA,
            pltpu.SemaphoreType.DMA,
            pltpu.SemaphoreType.DMA,
        ],
    )
    part3 = s3(h2, src_e, dst_e, a_e, q_e, pk)

    A, B, AB = pl.pallas_call(
        _t2_body,
        grid=(NP // BLK,),
        in_specs=[
            pl.BlockSpec((BLK, H), lambda i: (i, 0)),
            pl.BlockSpec((2, BLK, H), lambda i: (0, i, 0)),
            pl.BlockSpec((4, H), lambda i: (0, 0)),
            pl.BlockSpec((H, H), lambda i: (0, 0)),
            pl.BlockSpec((H, DD), lambda i: (0, 0)),
            pl.BlockSpec((H, DD), lambda i: (0, 0)),
            pl.BlockSpec((1, DD), lambda i: (0, 0)),
            pl.BlockSpec((H, 2), lambda i: (0, 0)),
            pl.BlockSpec((1, 2), lambda i: (0, 0)),
        ],
        out_specs=(pl.BlockSpec((BLK, DD), lambda i: (i, 0)),
                   pl.BlockSpec((BLK, DD), lambda i: (i, 0)),
                   pl.BlockSpec((BLK, 2), lambda i: (i, 0))),
        out_shape=(jax.ShapeDtypeStruct((NP, DD), f32),
                   jax.ShapeDtypeStruct((NP, DD), f32),
                   jax.ShapeDtypeStruct((NP, 2), f32)),
    )(h2, part3, bt, nn3_W, wa_m, wb_m, ba_m, wc, bc)

    s4 = pl.kernel(
        _s4_body,
        out_type=jax.ShapeDtypeStruct((E,), f32),
        mesh=_mesh,
        compiler_params=_scp,
        scratch_types=[
            pltpu.VMEM((BE,), i32),
            pltpu.VMEM((BE,), i32),
            pltpu.VMEM((2, CH), i32),
            pltpu.VMEM((2, CH), i32),
            pltpu.VMEM((BE,), f32),
            pltpu.VMEM((NP,), f32),
            pltpu.VMEM((NP,), f32),
            pltpu.VMEM((2 * CH, DD), f32),
            pltpu.VMEM((2 * CH, DD), f32),
            pltpu.VMEM((BE,), f32),
            pltpu.VMEM((4, 16), f32),
            pltpu.VMEM((2, DD), f32),
            pltpu.VMEM((4, 16), f32),
            pltpu.SemaphoreType.DMA,
            pltpu.SemaphoreType.DMA,
            pltpu.SemaphoreType.DMA,
            pltpu.SemaphoreType.DMA,
        ],
    )
    out = s4(A, B, AB.T, src_e, dst_e, q_e, cd, sc4)
    return out.reshape(E, 1)
